# Initial kernel scaffold; baseline (speedup 1.0000x reference)
import jax
import jax.numpy as jnp
from jax.experimental import pallas as pl


def kernel(x, edge_index, edge_attr, y, W0, a_src0, a_dst0, b0, W1, a_src1, a_dst1, b1, Wg, bg, W3, a_src3, a_dst3, b3):
    raise NotImplementedError("write your pallas kernel here")


# speedup vs baseline: 24.9819x; 24.9819x over previous
import functools

import jax
import jax.numpy as jnp
from jax import lax
from jax.experimental import pallas as pl
from jax.experimental.pallas import tpu as pltpu
from jax.experimental.pallas import tpu_sc as plsc

N = 100000
T_PAST = 12
HID = 32
TEMP = T_PAST + 2 * HID

BR = 512
NPAD = 196 * BR
NG = NPAD // BR
RPT = NPAD // 16
ZR = 128

E = 1600000
EE = E + N
BLK = 512
EPT = 53248
EPAD = 32 * EPT
NBLK = EPT // BLK
HALF_E = EPAD // 2

_MESH = dict(core_axis_name="c", subcore_axis_name="s",
             num_cores=2, num_subcores=16)

f32 = jnp.float32
i32 = jnp.int32


def _edge_base(i):
    c = lax.axis_index("c")
    s = lax.axis_index("s")
    return c * HALF_E + s * EPT + i * BLK


def _fill_idx(idxd, dst_v):
    for k in range(32):
        idxd[k // 8, pl.ds((k % 8) * 16, 16)] = dst_v[pl.ds(k * 16, 16)]


def _zero_rows(zbuf, acc):
    s = lax.axis_index("s")

    @pl.loop(0, RPT // ZR)
    def _(i):
        pltpu.sync_copy(zbuf, acc.at[pl.ds(s * RPT + i * ZR, ZR), :])


def _zero_rows1(zd, acc1):
    s = lax.axis_index("s")

    @pl.loop(0, RPT // ZR)
    def _(i):
        pltpu.sync_copy(zd, acc1.at[pl.ds(s * RPT + i * ZR, ZR)])


def _init_zbufs(zbuf, zd):
    zro = jnp.zeros((16,), f32)

    @pl.loop(0, ZR)
    def _(i):
        zbuf[i] = zro

    @pl.loop(0, ZR // 16)
    def _(i):
        zd[pl.ds(i * 16, 16)] = zro


def _flush(acc, out, row0):
    s = lax.axis_index("s")
    pltpu.sync_copy(acc.at[pl.ds(s * RPT, RPT), :],
                    out.at[pl.ds(row0 + s * RPT, RPT), :])


def _flush1(acc1, out1, row0):
    s = lax.axis_index("s")
    pltpu.sync_copy(acc1.at[pl.ds(s * RPT, RPT)],
                    out1.at[pl.ds(row0 + s * RPT, RPT)])


def _deg_body(dst_h, w_h, deg_out, dst_v, w_v, idxd, zbuf, zd, acc1, sem):
    c = lax.axis_index("c")
    _init_zbufs(zbuf, zd)
    _zero_rows1(zd, acc1)
    plsc.subcore_barrier()

    @pl.loop(0, NBLK)
    def _(i):
        be = _edge_base(i)
        pltpu.sync_copy(dst_h.at[pl.ds(be, BLK)], dst_v)
        pltpu.sync_copy(w_h.at[pl.ds(be, BLK)], w_v)
        _fill_idx(idxd, dst_v)
        for j in range(4):
            pltpu.sync_copy(w_v.at[pl.ds(j * 128, 128)],
                            acc1.at[idxd.at[j]], add=True)

    plsc.subcore_barrier()
    _flush1(acc1, deg_out, c * NPAD)


def _norm_body(src_h, dst_h, w_h, dinv_h, norm_out,
               src_v, dst_v, w_v, dv, dd, nv, sem):
    @pl.loop(0, NBLK)
    def _(i):
        be = _edge_base(i)
        pltpu.sync_copy(src_h.at[pl.ds(be, BLK)], src_v)
        pltpu.sync_copy(dst_h.at[pl.ds(be, BLK)], dst_v)
        pltpu.sync_copy(w_h.at[pl.ds(be, BLK)], w_v)
        pltpu.async_copy(dinv_h.at[src_v], dv, sem).wait()
        pltpu.async_copy(dinv_h.at[dst_v], dd, sem).wait()
        for k in range(32):
            sl = pl.ds(k * 16, 16)
            nv[sl] = dv[sl] * w_v[sl] * dd[sl]
        pltpu.sync_copy(nv, norm_out.at[pl.ds(be, BLK)])


def _gat_body(hlo, hhi, hs_h, hd_h, m_h, src_h, dst_h,
              num_out, den_out, ebuf,
              src_v, dst_v, hsv, hdv, ev, rows_v, idxd, zbuf, zd, m_v,
              acc, dacc, sem):
    c = lax.axis_index("c")
    _init_zbufs(zbuf, zd)
    pltpu.sync_copy(m_h, m_v)
    mv = m_v[...]

    for f, tab in enumerate((hlo, hhi)):
        _zero_rows(zbuf, acc)
        if f == 0:
            _zero_rows1(zd, dacc)
        plsc.subcore_barrier()

        @pl.loop(0, NBLK)
        def _(i):
            be = _edge_base(i)
            pltpu.sync_copy(src_h.at[pl.ds(be, BLK)], src_v)
            pltpu.sync_copy(dst_h.at[pl.ds(be, BLK)], dst_v)
            pltpu.async_copy(tab.at[src_v], rows_v, sem).wait()
            if f == 0:
                pltpu.async_copy(hs_h.at[src_v], hsv, sem).wait()
                pltpu.async_copy(hd_h.at[dst_v], hdv, sem).wait()
                for k in range(32):
                    sl = pl.ds(k * 16, 16)
                    a = hsv[sl] + hdv[sl]
                    a = jnp.where(a > 0, a, 0.2 * a)
                    ev[sl] = jnp.exp(a - mv)
                pltpu.sync_copy(ev, ebuf.at[pl.ds(be, BLK)])
            else:
                pltpu.sync_copy(ebuf.at[pl.ds(be, BLK)], ev)

            @pl.loop(0, BLK // 16)
            def _(k):
                e16 = ev[pl.ds(k * 16, 16)]
                for u in range(16):
                    t = k * 16 + u
                    rows_v[t] = rows_v[t] * e16[u]

            _fill_idx(idxd, dst_v)
            for j in range(4):
                pltpu.sync_copy(rows_v.at[pl.ds(j * 128, 128), :],
                                acc.at[idxd.at[j]], add=True)
                if f == 0:
                    pltpu.sync_copy(ev.at[pl.ds(j * 128, 128)],
                                    dacc.at[idxd.at[j]], add=True)

        plsc.subcore_barrier()
        _flush(acc, num_out, (c * 2 + f) * NPAD)
        if f == 0:
            _flush1(dacc, den_out, c * NPAD)
        plsc.subcore_barrier()


def _gcn_body(t0, t1, t2, t3, t4, src_h, dst_h, norm_h, acc_out,
              src_v, dst_v, nv, rows_v, idxd, zbuf, acc, sem):
    c = lax.axis_index("c")
    zro = jnp.zeros((16,), f32)

    @pl.loop(0, ZR)
    def _(i):
        zbuf[i] = zro

    for p, tab in enumerate((t0, t1, t2, t3, t4)):
        _zero_rows(zbuf, acc)
        plsc.subcore_barrier()

        @pl.loop(0, NBLK)
        def _(i):
            be = _edge_base(i)
            pltpu.sync_copy(src_h.at[pl.ds(be, BLK)], src_v)
            pltpu.sync_copy(dst_h.at[pl.ds(be, BLK)], dst_v)
            pltpu.sync_copy(norm_h.at[pl.ds(be, BLK)], nv)
            pltpu.async_copy(tab.at[src_v], rows_v, sem).wait()

            @pl.loop(0, BLK // 16)
            def _(k):
                e16 = nv[pl.ds(k * 16, 16)]
                for u in range(16):
                    t = k * 16 + u
                    rows_v[t] = rows_v[t] * e16[u]

            _fill_idx(idxd, dst_v)
            for j in range(4):
                pltpu.sync_copy(rows_v.at[pl.ds(j * 128, 128), :],
                                acc.at[idxd.at[j]], add=True)

        plsc.subcore_barrier()
        _flush(acc, acc_out, (c * 5 + p) * NPAD)
        plsc.subcore_barrier()


def _gat1_body(hs_h, hd_h, h_h, m_h, src_h, dst_h, num_out, den_out,
               src_v, dst_v, hsv, hdv, hv, ev, pv, idxd, zd, nacc, dacc,
               m_v, sem):
    c = lax.axis_index("c")
    zro = jnp.zeros((16,), f32)

    @pl.loop(0, ZR // 16)
    def _(i):
        zd[pl.ds(i * 16, 16)] = zro

    pltpu.sync_copy(m_h, m_v)
    mv = m_v[...]
    _zero_rows1(zd, nacc)
    _zero_rows1(zd, dacc)
    plsc.subcore_barrier()

    @pl.loop(0, NBLK)
    def _(i):
        be = _edge_base(i)
        pltpu.sync_copy(src_h.at[pl.ds(be, BLK)], src_v)
        pltpu.sync_copy(dst_h.at[pl.ds(be, BLK)], dst_v)
        pltpu.async_copy(hs_h.at[src_v], hsv, sem).wait()
        pltpu.async_copy(hd_h.at[dst_v], hdv, sem).wait()
        pltpu.async_copy(h_h.at[src_v], hv, sem).wait()
        for k in range(32):
            sl = pl.ds(k * 16, 16)
            a = hsv[sl] + hdv[sl]
            a = jnp.where(a > 0, a, 0.2 * a)
            e = jnp.exp(a - mv)
            ev[sl] = e
            pv[sl] = e * hv[sl]
        _fill_idx(idxd, dst_v)
        for j in range(4):
            pltpu.sync_copy(ev.at[pl.ds(j * 128, 128)],
                            dacc.at[idxd.at[j]], add=True)
            pltpu.sync_copy(pv.at[pl.ds(j * 128, 128)],
                            nacc.at[idxd.at[j]], add=True)

    plsc.subcore_barrier()
    _flush1(nacc, num_out, c * NPAD)
    _flush1(dacc, den_out, c * NPAD)


def _sc_kernel(body, out_shapes, scratch):
    mesh = plsc.VectorSubcoreMesh(**_MESH)
    return pl.kernel(body, out_type=out_shapes, mesh=mesh,
                     scratch_types=scratch,
                     compiler_params=pltpu.CompilerParams(
                         use_tc_tiling_on_sc=False))


def _sc_deg(dst, w):
    return _sc_kernel(
        _deg_body,
        [jax.ShapeDtypeStruct((2 * NPAD,), f32)],
        [pltpu.VMEM((BLK,), i32), pltpu.VMEM((BLK,), f32),
         pltpu.VMEM((4, 128), i32), pltpu.VMEM((ZR, 16), f32),
         pltpu.VMEM((ZR,), f32), pltpu.VMEM_SHARED((NPAD,), f32),
         pltpu.SemaphoreType.DMA],
    )(dst, w)[0]


def _sc_norm(src, dst, w, dinv):
    return _sc_kernel(
        _norm_body,
        [jax.ShapeDtypeStruct((EPAD,), f32)],
        [pltpu.VMEM((BLK,), i32), pltpu.VMEM((BLK,), i32),
         pltpu.VMEM((BLK,), f32), pltpu.VMEM((BLK,), f32),
         pltpu.VMEM((BLK,), f32), pltpu.VMEM((BLK,), f32),
         pltpu.SemaphoreType.DMA],
    )(src, dst, w, dinv)[0]


def _sc_gat(hlo, hhi, hs, hd, mvec, src, dst):
    outs = _sc_kernel(
        _gat_body,
        [jax.ShapeDtypeStruct((4 * NPAD, 16), f32),
         jax.ShapeDtypeStruct((2 * NPAD,), f32),
         jax.ShapeDtypeStruct((EPAD,), f32)],
        [pltpu.VMEM((BLK,), i32), pltpu.VMEM((BLK,), i32),
         pltpu.VMEM((BLK,), f32), pltpu.VMEM((BLK,), f32),
         pltpu.VMEM((BLK,), f32), pltpu.VMEM((BLK, 16), f32),
         pltpu.VMEM((4, 128), i32), pltpu.VMEM((ZR, 16), f32),
         pltpu.VMEM((ZR,), f32), pltpu.VMEM((16,), f32),
         pltpu.VMEM_SHARED((NPAD, 16), f32),
         pltpu.VMEM_SHARED((NPAD,), f32),
         pltpu.SemaphoreType.DMA],
    )(hlo, hhi, hs, hd, mvec, src, dst)
    return outs[0], outs[1]


def _sc_gcn(tabs, src, dst, norm):
    return _sc_kernel(
        _gcn_body,
        [jax.ShapeDtypeStruct((10 * NPAD, 16), f32)],
        [pltpu.VMEM((BLK,), i32), pltpu.VMEM((BLK,), i32),
         pltpu.VMEM((BLK,), f32), pltpu.VMEM((BLK, 16), f32),
         pltpu.VMEM((4, 128), i32), pltpu.VMEM((ZR, 16), f32),
         pltpu.VMEM_SHARED((NPAD, 16), f32),
         pltpu.SemaphoreType.DMA],
    )(*tabs, src, dst, norm)[0]


def _sc_gat1(hs, hd, h, mvec, src, dst):
    outs = _sc_kernel(
        _gat1_body,
        [jax.ShapeDtypeStruct((2 * NPAD,), f32),
         jax.ShapeDtypeStruct((2 * NPAD,), f32)],
        [pltpu.VMEM((BLK,), i32), pltpu.VMEM((BLK,), i32),
         pltpu.VMEM((BLK,), f32), pltpu.VMEM((BLK,), f32),
         pltpu.VMEM((BLK,), f32), pltpu.VMEM((BLK,), f32),
         pltpu.VMEM((BLK,), f32), pltpu.VMEM((4, 128), i32),
         pltpu.VMEM((ZR,), f32), pltpu.VMEM_SHARED((NPAD,), f32),
         pltpu.VMEM_SHARED((NPAD,), f32), pltpu.VMEM((16,), f32),
         pltpu.SemaphoreType.DMA],
    )(hs, hd, h, mvec, src, dst)
    return outs[0], outs[1]


def _row_spec(d):
    return pl.BlockSpec((BR, d), lambda r: (r, 0))


def _full_spec(shape):
    nd = len(shape)
    return pl.BlockSpec(shape, lambda r: (0,) * nd)


def _lead_spec(lead, d):
    return pl.BlockSpec((lead, BR, d), lambda r: (0, r, 0))


def _tc_call(body, in_specs, out_shapes, out_specs):
    return pl.pallas_call(
        body, grid=(NG,), in_specs=in_specs,
        out_shape=[jax.ShapeDtypeStruct(s, f32) for s in out_shapes],
        out_specs=out_specs)


def _tca_body(o0_ref, w_ref, as_ref, ad_ref, hlo_ref, hhi_ref,
              hs_ref, hd_ref):
    h = jnp.dot(o0_ref[...], w_ref[...], preferred_element_type=f32)
    hlo_ref[...] = h[:, :16]
    hhi_ref[...] = h[:, 16:32]
    hs_ref[...] = jnp.dot(h, as_ref[...], preferred_element_type=f32)
    hd_ref[...] = jnp.dot(h, ad_ref[...], preferred_element_type=f32)


def _tca(o0, W, a_s, a_d, k):
    return _tc_call(
        _tca_body,
        [_row_spec(k), _full_spec((k, HID)), _full_spec((HID, 1)),
         _full_spec((HID, 1))],
        [(NPAD, 16), (NPAD, 16), (NPAD, 1), (NPAD, 1)],
        [_row_spec(16), _row_spec(16), _row_spec(1), _row_spec(1)])(
            o0, W, a_s.reshape(HID, 1), a_d.reshape(HID, 1))


def _dinv_body(deg_ref, dinv_ref):
    d = deg_ref[...]
    dt = d[0] + d[1]
    dinv_ref[...] = jnp.where(dt > 0, lax.rsqrt(dt), 0.0)


def _dinv(deg2):
    return _tc_call(
        _dinv_body, [_lead_spec(2, 1)], [(NPAD, 1)], [_row_spec(1)])(
            deg2.reshape(2, NPAD, 1))[0]


def _tcb_body(num_ref, den_ref, b_ref, w_ref, as_ref, ad_ref,
              o1_ref, hlo_ref, hhi_ref, hs_ref, hd_ref):
    n = num_ref[...]
    dn = den_ref[...]
    numv = jnp.concatenate([n[0] + n[2], n[1] + n[3]], axis=1)
    den = dn[0] + dn[1]
    o1 = numv / (den + 1e-16) + b_ref[...]
    o1_ref[...] = o1
    h = jnp.dot(o1, w_ref[...], preferred_element_type=f32)
    hlo_ref[...] = h[:, :16]
    hhi_ref[...] = h[:, 16:32]
    hs_ref[...] = jnp.dot(h, as_ref[...], preferred_element_type=f32)
    hd_ref[...] = jnp.dot(h, ad_ref[...], preferred_element_type=f32)


def _tcb(num, den, b, W, a_s, a_d):
    return _tc_call(
        _tcb_body,
        [_lead_spec(4, 16), _lead_spec(2, 1), _full_spec((1, HID)),
         _full_spec((HID, HID)), _full_spec((HID, 1)),
         _full_spec((HID, 1))],
        [(NPAD, HID), (NPAD, 16), (NPAD, 16), (NPAD, 1), (NPAD, 1)],
        [_row_spec(HID), _row_spec(16), _row_spec(16), _row_spec(1),
         _row_spec(1)])(
            num.reshape(4, NPAD, 16), den.reshape(2, NPAD, 1),
            b.reshape(1, HID), W, a_s.reshape(HID, 1),
            a_d.reshape(HID, 1))


def _tcc_body(num_ref, den_ref, b_ref, o0_ref, o1_ref, wg_ref,
              h0_ref, h1_ref, h2_ref, h3_ref, h4_ref):
    n = num_ref[...]
    dn = den_ref[...]
    numv = jnp.concatenate([n[0] + n[2], n[1] + n[3]], axis=1)
    o2 = numv / (dn[0] + dn[1] + 1e-16) + b_ref[...]
    temp = jnp.concatenate([o0_ref[...], o1_ref[...], o2], axis=1)
    ht = jnp.dot(temp, wg_ref[...], preferred_element_type=f32)
    h0_ref[...] = ht[:, 0:16]
    h1_ref[...] = ht[:, 16:32]
    h2_ref[...] = ht[:, 32:48]
    h3_ref[...] = ht[:, 48:64]
    h4_ref[...] = jnp.concatenate(
        [ht[:, 64:76], jnp.zeros((BR, 4), f32)], axis=1)


def _tcc(num, den, b, o0, o1, Wg):
    return _tc_call(
        _tcc_body,
        [_lead_spec(4, 16), _lead_spec(2, 1), _full_spec((1, HID)),
         _row_spec(T_PAST), _row_spec(HID), _full_spec((TEMP, TEMP))],
        [(NPAD, 16)] * 5,
        [_row_spec(16)] * 5)(
            num.reshape(4, NPAD, 16), den.reshape(2, NPAD, 1),
            b.reshape(1, HID), o0, o1, Wg)


def _tcd_body(acc_ref, bg_ref, w3_ref, as_ref, ad_ref,
              h3_ref, hs_ref, hd_ref):
    a = acc_ref[...]
    g = jnp.concatenate([a[p] + a[5 + p] for p in range(5)], axis=1)
    xt = g[:, :TEMP] + bg_ref[...]
    h3 = jnp.dot(xt, w3_ref[...], preferred_element_type=f32)
    h3_ref[...] = h3
    hs_ref[...] = h3 * as_ref[0, 0]
    hd_ref[...] = h3 * ad_ref[0, 0]


def _tcd(accg, bg, W3, a_s3, a_d3):
    return _tc_call(
        _tcd_body,
        [_lead_spec(10, 16), _full_spec((1, TEMP)),
         _full_spec((TEMP, 1)), _full_spec((1, 1)), _full_spec((1, 1))],
        [(NPAD, 1)] * 3,
        [_row_spec(1)] * 3)(
            accg.reshape(10, NPAD, 16), bg.reshape(1, TEMP), W3,
            a_s3.reshape(1, 1), a_d3.reshape(1, 1))


def _tce_body(num_ref, den_ref, b_ref, o0_ref, yp_ref, o0n_ref):
    n = num_ref[...]
    dn = den_ref[...]
    yp = (n[0] + n[1]) / (dn[0] + dn[1] + 1e-16) + b_ref[...]
    yp = jnp.clip(yp, 0.0, 90.0)
    yp_ref[...] = yp
    o0n_ref[...] = jnp.concatenate([o0_ref[...][:, 1:], yp], axis=1)


def _tce(num, den, b3, o0):
    return _tc_call(
        _tce_body,
        [_lead_spec(2, 1), _lead_spec(2, 1), _full_spec((1, 1)),
         _row_spec(T_PAST)],
        [(NPAD, 1), (NPAD, T_PAST)],
        [_row_spec(1), _row_spec(T_PAST)])(
            num.reshape(2, NPAD, 1), den.reshape(2, NPAD, 1),
            b3.reshape(1, 1), o0)


def _leaky(v):
    return jnp.where(v > 0, v, 0.2 * v)


def _mvec(hs, hd):
    m = _leaky(jnp.max(hs[:N, 0]) + jnp.max(hd[:N, 0]))
    return jnp.full((16,), m, f32)


def kernel(x, edge_index, edge_attr, y, W0, a_src0, a_dst0, b0,
           W1, a_src1, a_dst1, b1, Wg, bg, W3, a_src3, a_dst3, b3):
    t_future = y.shape[1]
    loop = jnp.arange(N, dtype=edge_index.dtype)
    padn = EPAD - EE
    src = jnp.concatenate(
        [edge_index[0], loop, jnp.zeros((padn,), edge_index.dtype)])
    dst = jnp.concatenate(
        [edge_index[1], loop, jnp.full((padn,), N, edge_index.dtype)])
    w = jnp.concatenate(
        [edge_attr, jnp.ones((N,), f32), jnp.zeros((padn,), f32)])

    deg2 = _sc_deg(dst, w)
    dinv = _dinv(deg2)
    norm = _sc_norm(src, dst, w, dinv.reshape(NPAD))

    o0 = jnp.concatenate(
        [x, jnp.zeros((NPAD - N, T_PAST), f32)], axis=0)
    preds = []
    for _step in range(t_future):
        hlo, hhi, hs0, hd0 = _tca(o0, W0, a_src0, a_dst0, T_PAST)
        num0, den0 = _sc_gat(hlo, hhi, hs0.reshape(NPAD),
                             hd0.reshape(NPAD), _mvec(hs0, hd0), src, dst)
        o1, h1lo, h1hi, hs1, hd1 = _tcb(num0, den0, b0, W1,
                                        a_src1, a_dst1)
        num1, den1 = _sc_gat(h1lo, h1hi, hs1.reshape(NPAD),
                             hd1.reshape(NPAD), _mvec(hs1, hd1), src, dst)
        tabs = _tcc(num1, den1, b1, o0, o1, Wg)
        accg = _sc_gcn(tabs, src, dst, norm)
        h3, hs3, hd3 = _tcd(accg, bg, W3, a_src3, a_dst3)
        num3, den3 = _sc_gat1(hs3.reshape(NPAD), hd3.reshape(NPAD),
                              h3.reshape(NPAD), _mvec(hs3, hd3), src, dst)
        yp, o0 = _tce(num3, den3, b3, o0)
        preds.append(yp[:N])

    return jnp.concatenate(preds, axis=1)


# speedup vs baseline: 33.2904x; 1.3326x over previous
import functools

import jax
import jax.numpy as jnp
from jax import lax
from jax.experimental import pallas as pl
from jax.experimental.pallas import tpu as pltpu
from jax.experimental.pallas import tpu_sc as plsc

N = 100000
T_PAST = 12
HID = 32
TEMP = T_PAST + 2 * HID

BR = 512
NPAD = 196 * BR
NG = NPAD // BR
RPT = NPAD // 16
ZR = 64

E = 1600000
EE = E + N
BLK = 512
EPT = 53248
EPAD = 32 * EPT
NBLK = EPT // BLK
HALF_E = EPAD // 2

_MESH = dict(core_axis_name="c", subcore_axis_name="s",
             num_cores=2, num_subcores=16)

f32 = jnp.float32
i32 = jnp.int32


def _edge_base(i):
    c = lax.axis_index("c")
    s = lax.axis_index("s")
    return c * HALF_E + s * EPT + i * BLK


def _fill_idx(idxd, dst_v):
    for k in range(32):
        idxd[k // 8, pl.ds((k % 8) * 16, 16)] = dst_v[pl.ds(k * 16, 16)]


def _zero_rows(zbuf, acc):
    s = lax.axis_index("s")

    @pl.loop(0, RPT // ZR)
    def _(i):
        pltpu.sync_copy(zbuf, acc.at[pl.ds(s * RPT + i * ZR, ZR), :])


def _zero_rows1(zd, acc1):
    s = lax.axis_index("s")

    @pl.loop(0, RPT // ZR)
    def _(i):
        pltpu.sync_copy(zd, acc1.at[pl.ds(s * RPT + i * ZR, ZR)])


def _init_zbufs(zbuf, zd):
    zro = jnp.zeros((16,), f32)

    @pl.loop(0, ZR)
    def _(i):
        zbuf[i] = zro

    @pl.loop(0, ZR // 16)
    def _(i):
        zd[pl.ds(i * 16, 16)] = zro


def _flush(acc, out, row0):
    s = lax.axis_index("s")
    pltpu.sync_copy(acc.at[pl.ds(s * RPT, RPT), :],
                    out.at[pl.ds(row0 + s * RPT, RPT), :])


def _flush1(acc1, out1, row0):
    s = lax.axis_index("s")
    pltpu.sync_copy(acc1.at[pl.ds(s * RPT, RPT)],
                    out1.at[pl.ds(row0 + s * RPT, RPT)])


def _deg_body(dst_h, w_h, deg_out, dst_v, w_v, idxd, zbuf, zd, acc1, sem):
    c = lax.axis_index("c")
    _init_zbufs(zbuf, zd)
    _zero_rows1(zd, acc1)
    plsc.subcore_barrier()

    @pl.loop(0, NBLK)
    def _(i):
        be = _edge_base(i)
        pltpu.sync_copy(dst_h.at[pl.ds(be, BLK)], dst_v)
        pltpu.sync_copy(w_h.at[pl.ds(be, BLK)], w_v)
        _fill_idx(idxd, dst_v)
        for j in range(4):
            pltpu.sync_copy(w_v.at[pl.ds(j * 128, 128)],
                            acc1.at[idxd.at[j]], add=True)

    plsc.subcore_barrier()
    _flush1(acc1, deg_out, c * NPAD)


def _norm_body(src_h, dst_h, w_h, dinv_h, norm_out,
               src_v, dst_v, w_v, dv, dd, nv, sem):
    @pl.loop(0, NBLK)
    def _(i):
        be = _edge_base(i)
        pltpu.sync_copy(src_h.at[pl.ds(be, BLK)], src_v)
        pltpu.sync_copy(dst_h.at[pl.ds(be, BLK)], dst_v)
        pltpu.sync_copy(w_h.at[pl.ds(be, BLK)], w_v)
        pltpu.async_copy(dinv_h.at[src_v], dv, sem).wait()
        pltpu.async_copy(dinv_h.at[dst_v], dd, sem).wait()
        for k in range(32):
            sl = pl.ds(k * 16, 16)
            nv[sl] = dv[sl] * w_v[sl] * dd[sl]
        pltpu.sync_copy(nv, norm_out.at[pl.ds(be, BLK)])


GB = 1024


def _batch_base(i):
    c = lax.axis_index("c")
    s = lax.axis_index("s")
    return c * HALF_E + s * EPT + i * GB


def _gat_body(hlo, hhi, hs_h, hd_h, m_h, src_h, dst2_h,
              num_out, den_out, ebuf,
              src_v, dst1_v, dst2_v, hsv, hdv, ev, rows_v, zbuf, zd, m_v,
              acc, dacc, semL, semR, semA, semB, semS):
    c = lax.axis_index("c")
    _init_zbufs(zbuf, zd)
    pltpu.sync_copy(m_h, m_v)
    mv = m_v[...]

    for f, tab in enumerate((hlo, hhi)):
        _zero_rows(zbuf, acc)
        if f == 0:
            _zero_rows1(zd, dacc)
        plsc.subcore_barrier()

        @pl.loop(0, NBLK // 2)
        def _(i):
            be = _batch_base(i)
            br = be // 128
            cs = pltpu.async_copy(src_h.at[pl.ds(be, GB)], src_v, semL)
            cd = pltpu.async_copy(dst2_h.at[pl.ds(br, GB // 128), :],
                                  dst2_v, semL)
            cs.wait()
            cd.wait()
            gr = pltpu.async_copy(tab.at[src_v], rows_v, semR)
            if f == 0:
                for j in range(GB // 128):
                    for m in range(8):
                        dst1_v[pl.ds(j * 128 + m * 16, 16)] = (
                            dst2_v[j, pl.ds(m * 16, 16)])
                ga = pltpu.async_copy(hs_h.at[src_v], hsv, semA)
                gb = pltpu.async_copy(hd_h.at[dst1_v], hdv, semB)
                ga.wait()
                gb.wait()
                for k in range(GB // 16):
                    sl = pl.ds(k * 16, 16)
                    a = hsv[sl] + hdv[sl]
                    a = jnp.where(a > 0, a, 0.2 * a)
                    ev[sl] = jnp.exp(a - mv)
                ce = pltpu.async_copy(ev, ebuf.at[pl.ds(be, GB)], semL)
            else:
                pltpu.async_copy(ebuf.at[pl.ds(be, GB)], ev, semL).wait()
            gr.wait()

            @pl.loop(0, GB // 16)
            def _(k):
                e16 = ev[pl.ds(k * 16, 16)]
                for u in range(16):
                    t = k * 16 + u
                    rows_v[t] = rows_v[t] * e16[u]

            sc = []
            for j in range(GB // 128):
                sc.append(pltpu.async_copy(
                    rows_v.at[pl.ds(j * 128, 128), :],
                    acc.at[dst2_v.at[j]], semS, add=True))
                if f == 0:
                    sc.append(pltpu.async_copy(
                        ev.at[pl.ds(j * 128, 128)],
                        dacc.at[dst2_v.at[j]], semS, add=True))
            for d in sc:
                d.wait()
            if f == 0:
                ce.wait()

        plsc.subcore_barrier()
        _flush(acc, num_out, (c * 2 + f) * NPAD)
        if f == 0:
            _flush1(dacc, den_out, c * NPAD)
        plsc.subcore_barrier()


def _gcn_body(t0, t1, t2, t3, t4, src_h, dst2_h, norm_h, acc_out,
              src_v, dst2_v, nv, rows_v, zbuf, acc,
              semL, semR, semS):
    c = lax.axis_index("c")
    zro = jnp.zeros((16,), f32)

    @pl.loop(0, ZR)
    def _(i):
        zbuf[i] = zro

    for p, tab in enumerate((t0, t1, t2, t3, t4)):
        _zero_rows(zbuf, acc)
        plsc.subcore_barrier()

        @pl.loop(0, NBLK // 2)
        def _(i):
            be = _batch_base(i)
            br = be // 128
            cs = pltpu.async_copy(src_h.at[pl.ds(be, GB)], src_v, semL)
            cd = pltpu.async_copy(dst2_h.at[pl.ds(br, GB // 128), :],
                                  dst2_v, semL)
            cn = pltpu.async_copy(norm_h.at[pl.ds(be, GB)], nv, semL)
            cs.wait()
            cd.wait()
            gr = pltpu.async_copy(tab.at[src_v], rows_v, semR)
            cn.wait()
            gr.wait()

            @pl.loop(0, GB // 16)
            def _(k):
                e16 = nv[pl.ds(k * 16, 16)]
                for u in range(16):
                    t = k * 16 + u
                    rows_v[t] = rows_v[t] * e16[u]

            sc = []
            for j in range(GB // 128):
                sc.append(pltpu.async_copy(
                    rows_v.at[pl.ds(j * 128, 128), :],
                    acc.at[dst2_v.at[j]], semS, add=True))
            for d in sc:
                d.wait()

        plsc.subcore_barrier()
        _flush(acc, acc_out, (c * 5 + p) * NPAD)
        plsc.subcore_barrier()


def _gat1_body(hs_h, hd_h, h_h, m_h, src_h, dst_h, num_out, den_out,
               src_v, dst_v, hsv, hdv, hv, ev, pv, idxd, zd, nacc, dacc,
               m_v, sem):
    c = lax.axis_index("c")
    zro = jnp.zeros((16,), f32)

    @pl.loop(0, ZR // 16)
    def _(i):
        zd[pl.ds(i * 16, 16)] = zro

    pltpu.sync_copy(m_h, m_v)
    mv = m_v[...]
    _zero_rows1(zd, nacc)
    _zero_rows1(zd, dacc)
    plsc.subcore_barrier()

    @pl.loop(0, NBLK)
    def _(i):
        be = _edge_base(i)
        pltpu.sync_copy(src_h.at[pl.ds(be, BLK)], src_v)
        pltpu.sync_copy(dst_h.at[pl.ds(be, BLK)], dst_v)
        pltpu.async_copy(hs_h.at[src_v], hsv, sem).wait()
        pltpu.async_copy(hd_h.at[dst_v], hdv, sem).wait()
        pltpu.async_copy(h_h.at[src_v], hv, sem).wait()
        for k in range(32):
            sl = pl.ds(k * 16, 16)
            a = hsv[sl] + hdv[sl]
            a = jnp.where(a > 0, a, 0.2 * a)
            e = jnp.exp(a - mv)
            ev[sl] = e
            pv[sl] = e * hv[sl]
        _fill_idx(idxd, dst_v)
        for j in range(4):
            pltpu.sync_copy(ev.at[pl.ds(j * 128, 128)],
                            dacc.at[idxd.at[j]], add=True)
            pltpu.sync_copy(pv.at[pl.ds(j * 128, 128)],
                            nacc.at[idxd.at[j]], add=True)

    plsc.subcore_barrier()
    _flush1(nacc, num_out, c * NPAD)
    _flush1(dacc, den_out, c * NPAD)


def _sc_kernel(body, out_shapes, scratch):
    mesh = plsc.VectorSubcoreMesh(**_MESH)
    return pl.kernel(body, out_type=out_shapes, mesh=mesh,
                     scratch_types=scratch,
                     compiler_params=pltpu.CompilerParams(
                         use_tc_tiling_on_sc=False))


def _sc_deg(dst, w):
    return _sc_kernel(
        _deg_body,
        [jax.ShapeDtypeStruct((2 * NPAD,), f32)],
        [pltpu.VMEM((BLK,), i32), pltpu.VMEM((BLK,), f32),
         pltpu.VMEM((4, 128), i32), pltpu.VMEM((ZR, 16), f32),
         pltpu.VMEM((ZR,), f32), pltpu.VMEM_SHARED((NPAD,), f32),
         pltpu.SemaphoreType.DMA],
    )(dst, w)[0]


def _sc_norm(src, dst, w, dinv):
    return _sc_kernel(
        _norm_body,
        [jax.ShapeDtypeStruct((EPAD,), f32)],
        [pltpu.VMEM((BLK,), i32), pltpu.VMEM((BLK,), i32),
         pltpu.VMEM((BLK,), f32), pltpu.VMEM((BLK,), f32),
         pltpu.VMEM((BLK,), f32), pltpu.VMEM((BLK,), f32),
         pltpu.SemaphoreType.DMA],
    )(src, dst, w, dinv)[0]


def _sc_gat(hlo, hhi, hs, hd, mvec, src, dst2):
    outs = _sc_kernel(
        _gat_body,
        [jax.ShapeDtypeStruct((4 * NPAD, 16), f32),
         jax.ShapeDtypeStruct((2 * NPAD,), f32),
         jax.ShapeDtypeStruct((EPAD,), f32)],
        [pltpu.VMEM((GB,), i32), pltpu.VMEM((GB,), i32),
         pltpu.VMEM((GB // 128, 128), i32),
         pltpu.VMEM((GB,), f32), pltpu.VMEM((GB,), f32),
         pltpu.VMEM((GB,), f32), pltpu.VMEM((GB, 16), f32),
         pltpu.VMEM((ZR, 16), f32),
         pltpu.VMEM((ZR,), f32), pltpu.VMEM((16,), f32),
         pltpu.VMEM_SHARED((NPAD, 16), f32),
         pltpu.VMEM_SHARED((NPAD,), f32),
         pltpu.SemaphoreType.DMA, pltpu.SemaphoreType.DMA,
         pltpu.SemaphoreType.DMA, pltpu.SemaphoreType.DMA,
         pltpu.SemaphoreType.DMA],
    )(hlo, hhi, hs, hd, mvec, src, dst2)
    return outs[0], outs[1]


def _sc_gcn(tabs, src, dst2, norm):
    return _sc_kernel(
        _gcn_body,
        [jax.ShapeDtypeStruct((10 * NPAD, 16), f32)],
        [pltpu.VMEM((GB,), i32), pltpu.VMEM((GB // 128, 128), i32),
         pltpu.VMEM((GB,), f32), pltpu.VMEM((GB, 16), f32),
         pltpu.VMEM((ZR, 16), f32),
         pltpu.VMEM_SHARED((NPAD, 16), f32),
         pltpu.SemaphoreType.DMA, pltpu.SemaphoreType.DMA,
         pltpu.SemaphoreType.DMA],
    )(*tabs, src, dst2, norm)[0]


def _sc_gat1(hs, hd, h, mvec, src, dst):
    outs = _sc_kernel(
        _gat1_body,
        [jax.ShapeDtypeStruct((2 * NPAD,), f32),
         jax.ShapeDtypeStruct((2 * NPAD,), f32)],
        [pltpu.VMEM((BLK,), i32), pltpu.VMEM((BLK,), i32),
         pltpu.VMEM((BLK,), f32), pltpu.VMEM((BLK,), f32),
         pltpu.VMEM((BLK,), f32), pltpu.VMEM((BLK,), f32),
         pltpu.VMEM((BLK,), f32), pltpu.VMEM((4, 128), i32),
         pltpu.VMEM((ZR,), f32), pltpu.VMEM_SHARED((NPAD,), f32),
         pltpu.VMEM_SHARED((NPAD,), f32), pltpu.VMEM((16,), f32),
         pltpu.SemaphoreType.DMA],
    )(hs, hd, h, mvec, src, dst)
    return outs[0], outs[1]


def _row_spec(d):
    return pl.BlockSpec((BR, d), lambda r: (r, 0))


def _full_spec(shape):
    nd = len(shape)
    return pl.BlockSpec(shape, lambda r: (0,) * nd)


def _lead_spec(lead, d):
    return pl.BlockSpec((lead, BR, d), lambda r: (0, r, 0))


def _tc_call(body, in_specs, out_shapes, out_specs):
    return pl.pallas_call(
        body, grid=(NG,), in_specs=in_specs,
        out_shape=[jax.ShapeDtypeStruct(s, f32) for s in out_shapes],
        out_specs=out_specs)


def _tca_body(o0_ref, w_ref, as_ref, ad_ref, hlo_ref, hhi_ref,
              hs_ref, hd_ref):
    h = jnp.dot(o0_ref[...], w_ref[...], preferred_element_type=f32)
    hlo_ref[...] = h[:, :16]
    hhi_ref[...] = h[:, 16:32]
    hs_ref[...] = jnp.dot(h, as_ref[...], preferred_element_type=f32)
    hd_ref[...] = jnp.dot(h, ad_ref[...], preferred_element_type=f32)


def _tca(o0, W, a_s, a_d, k):
    return _tc_call(
        _tca_body,
        [_row_spec(k), _full_spec((k, HID)), _full_spec((HID, 1)),
         _full_spec((HID, 1))],
        [(NPAD, 16), (NPAD, 16), (NPAD, 1), (NPAD, 1)],
        [_row_spec(16), _row_spec(16), _row_spec(1), _row_spec(1)])(
            o0, W, a_s.reshape(HID, 1), a_d.reshape(HID, 1))


def _dinv_body(deg_ref, dinv_ref):
    d = deg_ref[...]
    dt = d[0] + d[1]
    dinv_ref[...] = jnp.where(dt > 0, lax.rsqrt(dt), 0.0)


def _dinv(deg2):
    return _tc_call(
        _dinv_body, [_lead_spec(2, 1)], [(NPAD, 1)], [_row_spec(1)])(
            deg2.reshape(2, NPAD, 1))[0]


def _tcb_body(num_ref, den_ref, b_ref, w_ref, as_ref, ad_ref,
              o1_ref, hlo_ref, hhi_ref, hs_ref, hd_ref):
    n = num_ref[...]
    dn = den_ref[...]
    numv = jnp.concatenate([n[0] + n[2], n[1] + n[3]], axis=1)
    den = dn[0] + dn[1]
    o1 = numv / (den + 1e-16) + b_ref[...]
    o1_ref[...] = o1
    h = jnp.dot(o1, w_ref[...], preferred_element_type=f32)
    hlo_ref[...] = h[:, :16]
    hhi_ref[...] = h[:, 16:32]
    hs_ref[...] = jnp.dot(h, as_ref[...], preferred_element_type=f32)
    hd_ref[...] = jnp.dot(h, ad_ref[...], preferred_element_type=f32)


def _tcb(num, den, b, W, a_s, a_d):
    return _tc_call(
        _tcb_body,
        [_lead_spec(4, 16), _lead_spec(2, 1), _full_spec((1, HID)),
         _full_spec((HID, HID)), _full_spec((HID, 1)),
         _full_spec((HID, 1))],
        [(NPAD, HID), (NPAD, 16), (NPAD, 16), (NPAD, 1), (NPAD, 1)],
        [_row_spec(HID), _row_spec(16), _row_spec(16), _row_spec(1),
         _row_spec(1)])(
            num.reshape(4, NPAD, 16), den.reshape(2, NPAD, 1),
            b.reshape(1, HID), W, a_s.reshape(HID, 1),
            a_d.reshape(HID, 1))


def _tcc_body(num_ref, den_ref, b_ref, o0_ref, o1_ref, wg_ref,
              h0_ref, h1_ref, h2_ref, h3_ref, h4_ref):
    n = num_ref[...]
    dn = den_ref[...]
    numv = jnp.concatenate([n[0] + n[2], n[1] + n[3]], axis=1)
    o2 = numv / (dn[0] + dn[1] + 1e-16) + b_ref[...]
    temp = jnp.concatenate([o0_ref[...], o1_ref[...], o2], axis=1)
    ht = jnp.dot(temp, wg_ref[...], preferred_element_type=f32)
    h0_ref[...] = ht[:, 0:16]
    h1_ref[...] = ht[:, 16:32]
    h2_ref[...] = ht[:, 32:48]
    h3_ref[...] = ht[:, 48:64]
    h4_ref[...] = jnp.concatenate(
        [ht[:, 64:76], jnp.zeros((BR, 4), f32)], axis=1)


def _tcc(num, den, b, o0, o1, Wg):
    return _tc_call(
        _tcc_body,
        [_lead_spec(4, 16), _lead_spec(2, 1), _full_spec((1, HID)),
         _row_spec(T_PAST), _row_spec(HID), _full_spec((TEMP, TEMP))],
        [(NPAD, 16)] * 5,
        [_row_spec(16)] * 5)(
            num.reshape(4, NPAD, 16), den.reshape(2, NPAD, 1),
            b.reshape(1, HID), o0, o1, Wg)


def _tcd_body(acc_ref, bg_ref, w3_ref, as_ref, ad_ref,
              h3_ref, hs_ref, hd_ref):
    a = acc_ref[...]
    g = jnp.concatenate([a[p] + a[5 + p] for p in range(5)], axis=1)
    xt = g[:, :TEMP] + bg_ref[...]
    h3 = jnp.dot(xt, w3_ref[...], preferred_element_type=f32)
    h3_ref[...] = h3
    hs_ref[...] = h3 * as_ref[0, 0]
    hd_ref[...] = h3 * ad_ref[0, 0]


def _tcd(accg, bg, W3, a_s3, a_d3):
    return _tc_call(
        _tcd_body,
        [_lead_spec(10, 16), _full_spec((1, TEMP)),
         _full_spec((TEMP, 1)), _full_spec((1, 1)), _full_spec((1, 1))],
        [(NPAD, 1)] * 3,
        [_row_spec(1)] * 3)(
            accg.reshape(10, NPAD, 16), bg.reshape(1, TEMP), W3,
            a_s3.reshape(1, 1), a_d3.reshape(1, 1))


def _tce_body(num_ref, den_ref, b_ref, o0_ref, yp_ref, o0n_ref):
    n = num_ref[...]
    dn = den_ref[...]
    yp = (n[0] + n[1]) / (dn[0] + dn[1] + 1e-16) + b_ref[...]
    yp = jnp.clip(yp, 0.0, 90.0)
    yp_ref[...] = yp
    o0n_ref[...] = jnp.concatenate([o0_ref[...][:, 1:], yp], axis=1)


def _tce(num, den, b3, o0):
    return _tc_call(
        _tce_body,
        [_lead_spec(2, 1), _lead_spec(2, 1), _full_spec((1, 1)),
         _row_spec(T_PAST)],
        [(NPAD, 1), (NPAD, T_PAST)],
        [_row_spec(1), _row_spec(T_PAST)])(
            num.reshape(2, NPAD, 1), den.reshape(2, NPAD, 1),
            b3.reshape(1, 1), o0)


def _leaky(v):
    return jnp.where(v > 0, v, 0.2 * v)


def _mvec(hs, hd):
    m = _leaky(jnp.max(hs[:N, 0]) + jnp.max(hd[:N, 0]))
    return jnp.full((16,), m, f32)


def kernel(x, edge_index, edge_attr, y, W0, a_src0, a_dst0, b0,
           W1, a_src1, a_dst1, b1, Wg, bg, W3, a_src3, a_dst3, b3):
    t_future = y.shape[1]
    loop = jnp.arange(N, dtype=edge_index.dtype)
    padn = EPAD - EE
    src = jnp.concatenate(
        [edge_index[0], loop, jnp.zeros((padn,), edge_index.dtype)])
    dst = jnp.concatenate(
        [edge_index[1], loop, jnp.full((padn,), N, edge_index.dtype)])
    w = jnp.concatenate(
        [edge_attr, jnp.ones((N,), f32), jnp.zeros((padn,), f32)])

    dst2 = dst.reshape(EPAD // 128, 128)
    deg2 = _sc_deg(dst, w)
    dinv = _dinv(deg2)
    norm = _sc_norm(src, dst, w, dinv.reshape(NPAD))

    o0 = jnp.concatenate(
        [x, jnp.zeros((NPAD - N, T_PAST), f32)], axis=0)
    preds = []
    for _step in range(t_future):
        hlo, hhi, hs0, hd0 = _tca(o0, W0, a_src0, a_dst0, T_PAST)
        num0, den0 = _sc_gat(hlo, hhi, hs0.reshape(NPAD),
                             hd0.reshape(NPAD), _mvec(hs0, hd0),
                             src, dst2)
        o1, h1lo, h1hi, hs1, hd1 = _tcb(num0, den0, b0, W1,
                                        a_src1, a_dst1)
        num1, den1 = _sc_gat(h1lo, h1hi, hs1.reshape(NPAD),
                             hd1.reshape(NPAD), _mvec(hs1, hd1),
                             src, dst2)
        tabs = _tcc(num1, den1, b1, o0, o1, Wg)
        accg = _sc_gcn(tabs, src, dst2, norm)
        h3, hs3, hd3 = _tcd(accg, bg, W3, a_src3, a_dst3)
        num3, den3 = _sc_gat1(hs3.reshape(NPAD), hd3.reshape(NPAD),
                              h3.reshape(NPAD), _mvec(hs3, hd3), src, dst)
        yp, o0 = _tce(num3, den3, b3, o0)
        preds.append(yp[:N])

    return jnp.concatenate(preds, axis=1)


# speedup vs baseline: 37.8193x; 1.1360x over previous
import functools

import jax
import jax.numpy as jnp
from jax import lax
from jax.experimental import pallas as pl
from jax.experimental.pallas import tpu as pltpu
from jax.experimental.pallas import tpu_sc as plsc

N = 100000
T_PAST = 12
HID = 32
TEMP = T_PAST + 2 * HID

BR = 512
NPAD = 196 * BR
NG = NPAD // BR
RPT = NPAD // 16
ZR = 64

E = 1600000
EE = E + N
BLK = 512
EPT = 53248
EPAD = 32 * EPT
NBLK = EPT // BLK
HALF_E = EPAD // 2

_MESH = dict(core_axis_name="c", subcore_axis_name="s",
             num_cores=2, num_subcores=16)

f32 = jnp.float32
i32 = jnp.int32


def _edge_base(i):
    c = lax.axis_index("c")
    s = lax.axis_index("s")
    return c * HALF_E + s * EPT + i * BLK


def _fill_idx(idxd, dst_v):
    for k in range(32):
        idxd[k // 8, pl.ds((k % 8) * 16, 16)] = dst_v[pl.ds(k * 16, 16)]


def _zero_rows(zbuf, acc):
    s = lax.axis_index("s")

    @pl.loop(0, RPT // ZR)
    def _(i):
        pltpu.sync_copy(zbuf, acc.at[pl.ds(s * RPT + i * ZR, ZR), :])


def _zero_rows1(zd, acc1):
    s = lax.axis_index("s")

    @pl.loop(0, RPT // ZR)
    def _(i):
        pltpu.sync_copy(zd, acc1.at[pl.ds(s * RPT + i * ZR, ZR)])


def _init_zbufs(zbuf, zd):
    zro = jnp.zeros((16,), f32)

    @pl.loop(0, ZR)
    def _(i):
        zbuf[i] = zro

    @pl.loop(0, ZR // 16)
    def _(i):
        zd[pl.ds(i * 16, 16)] = zro


def _flush(acc, out, row0):
    s = lax.axis_index("s")
    pltpu.sync_copy(acc.at[pl.ds(s * RPT, RPT), :],
                    out.at[pl.ds(row0 + s * RPT, RPT), :])


def _flush1(acc1, out1, row0):
    s = lax.axis_index("s")
    pltpu.sync_copy(acc1.at[pl.ds(s * RPT, RPT)],
                    out1.at[pl.ds(row0 + s * RPT, RPT)])


def _deg_body(dst_h, w_h, deg_out, dst_v, w_v, idxd, zbuf, zd, acc1, sem):
    c = lax.axis_index("c")
    _init_zbufs(zbuf, zd)
    _zero_rows1(zd, acc1)
    plsc.subcore_barrier()

    @pl.loop(0, NBLK)
    def _(i):
        be = _edge_base(i)
        pltpu.sync_copy(dst_h.at[pl.ds(be, BLK)], dst_v)
        pltpu.sync_copy(w_h.at[pl.ds(be, BLK)], w_v)
        _fill_idx(idxd, dst_v)
        for j in range(4):
            pltpu.sync_copy(w_v.at[pl.ds(j * 128, 128)],
                            acc1.at[idxd.at[j]], add=True)

    plsc.subcore_barrier()
    _flush1(acc1, deg_out, c * NPAD)


def _norm_body(src_h, dst_h, w_h, dinv_h, norm_out,
               src_v, dst_v, w_v, dv, dd, nv, sem):
    @pl.loop(0, NBLK)
    def _(i):
        be = _edge_base(i)
        pltpu.sync_copy(src_h.at[pl.ds(be, BLK)], src_v)
        pltpu.sync_copy(dst_h.at[pl.ds(be, BLK)], dst_v)
        pltpu.sync_copy(w_h.at[pl.ds(be, BLK)], w_v)
        pltpu.async_copy(dinv_h.at[src_v], dv, sem).wait()
        pltpu.async_copy(dinv_h.at[dst_v], dd, sem).wait()
        for k in range(32):
            sl = pl.ds(k * 16, 16)
            nv[sl] = dv[sl] * w_v[sl] * dd[sl]
        pltpu.sync_copy(nv, norm_out.at[pl.ds(be, BLK)])


GB = 1024


def _batch_base(i):
    c = lax.axis_index("c")
    s = lax.axis_index("s")
    return c * HALF_E + s * EPT + i * GB


def _gat_body(hlo, hhi, hs_h, hd_h, m_h, src_h, dst2_h,
              num_out, den_out, ebuf,
              src_v, dst1_v, dst2_v, hsv, hdv, ev, rows_v, zbuf, zd, m_v,
              acc, dacc, semL, semR, semA, semB, semS):
    c = lax.axis_index("c")
    _init_zbufs(zbuf, zd)
    pltpu.sync_copy(m_h, m_v)
    mv = m_v[...]

    for f, tab in enumerate((hlo, hhi)):
        _zero_rows(zbuf, acc)
        if f == 0:
            _zero_rows1(zd, dacc)
        plsc.subcore_barrier()

        @pl.loop(0, NBLK // 2)
        def _(i):
            be = _batch_base(i)
            br = be // 128
            cs = pltpu.async_copy(src_h.at[pl.ds(be, GB)], src_v, semL)
            cd = pltpu.async_copy(dst2_h.at[pl.ds(br, GB // 128), :],
                                  dst2_v, semL)
            cs.wait()
            cd.wait()
            gr = pltpu.async_copy(tab.at[src_v], rows_v, semR)
            if f == 0:
                for j in range(GB // 128):
                    for m in range(8):
                        dst1_v[pl.ds(j * 128 + m * 16, 16)] = (
                            dst2_v[j, pl.ds(m * 16, 16)])
                ga = pltpu.async_copy(hs_h.at[src_v], hsv, semA)
                gb = pltpu.async_copy(hd_h.at[dst1_v], hdv, semB)
                ga.wait()
                gb.wait()
                for k in range(GB // 16):
                    sl = pl.ds(k * 16, 16)
                    a = hsv[sl] + hdv[sl]
                    a = jnp.where(a > 0, a, 0.2 * a)
                    ev[sl] = jnp.exp(a - mv)
                ce = pltpu.async_copy(ev, ebuf.at[pl.ds(be, GB)], semL)
            else:
                pltpu.async_copy(ebuf.at[pl.ds(be, GB)], ev, semL).wait()
            gr.wait()

            @pl.loop(0, GB // 16)
            def _(k):
                e16 = ev[pl.ds(k * 16, 16)]
                for u in range(16):
                    t = k * 16 + u
                    rows_v[t] = rows_v[t] * e16[u]

            sc = []
            for j in range(GB // 128):
                sc.append(pltpu.async_copy(
                    rows_v.at[pl.ds(j * 128, 128), :],
                    acc.at[dst2_v.at[j]], semS, add=True))
                if f == 0:
                    sc.append(pltpu.async_copy(
                        ev.at[pl.ds(j * 128, 128)],
                        dacc.at[dst2_v.at[j]], semS, add=True))
            for d in sc:
                d.wait()
            if f == 0:
                ce.wait()

        plsc.subcore_barrier()
        _flush(acc, num_out, (c * 2 + f) * NPAD)
        if f == 0:
            _flush1(dacc, den_out, c * NPAD)
        plsc.subcore_barrier()


def _gcn_body(t0, t1, t2, t3, t4, src_h, dst2_h, norm_h, acc_out,
              src_v, dst2_v, nv, rows_v, zbuf, acc,
              semL, semR, semS):
    c = lax.axis_index("c")
    zro = jnp.zeros((16,), f32)

    @pl.loop(0, ZR)
    def _(i):
        zbuf[i] = zro

    for p, tab in enumerate((t0, t1, t2, t3, t4)):
        _zero_rows(zbuf, acc)
        plsc.subcore_barrier()

        @pl.loop(0, NBLK // 2)
        def _(i):
            be = _batch_base(i)
            br = be // 128
            cs = pltpu.async_copy(src_h.at[pl.ds(be, GB)], src_v, semL)
            cd = pltpu.async_copy(dst2_h.at[pl.ds(br, GB // 128), :],
                                  dst2_v, semL)
            cn = pltpu.async_copy(norm_h.at[pl.ds(be, GB)], nv, semL)
            cs.wait()
            cd.wait()
            gr = pltpu.async_copy(tab.at[src_v], rows_v, semR)
            cn.wait()
            gr.wait()

            @pl.loop(0, GB // 16)
            def _(k):
                e16 = nv[pl.ds(k * 16, 16)]
                for u in range(16):
                    t = k * 16 + u
                    rows_v[t] = rows_v[t] * e16[u]

            sc = []
            for j in range(GB // 128):
                sc.append(pltpu.async_copy(
                    rows_v.at[pl.ds(j * 128, 128), :],
                    acc.at[dst2_v.at[j]], semS, add=True))
            for d in sc:
                d.wait()

        plsc.subcore_barrier()
        _flush(acc, acc_out, (c * 5 + p) * NPAD)
        plsc.subcore_barrier()


def _gat1_body(hs_h, hd_h, h_h, m_h, src_h, dst_h, num_out, den_out,
               src_v, dst_v, hsv, hdv, hv, ev, pv, idxd, zd, nacc, dacc,
               m_v, sem):
    c = lax.axis_index("c")
    zro = jnp.zeros((16,), f32)

    @pl.loop(0, ZR // 16)
    def _(i):
        zd[pl.ds(i * 16, 16)] = zro

    pltpu.sync_copy(m_h, m_v)
    mv = m_v[...]
    _zero_rows1(zd, nacc)
    _zero_rows1(zd, dacc)
    plsc.subcore_barrier()

    @pl.loop(0, NBLK)
    def _(i):
        be = _edge_base(i)
        pltpu.sync_copy(src_h.at[pl.ds(be, BLK)], src_v)
        pltpu.sync_copy(dst_h.at[pl.ds(be, BLK)], dst_v)
        pltpu.async_copy(hs_h.at[src_v], hsv, sem).wait()
        pltpu.async_copy(hd_h.at[dst_v], hdv, sem).wait()
        pltpu.async_copy(h_h.at[src_v], hv, sem).wait()
        for k in range(32):
            sl = pl.ds(k * 16, 16)
            a = hsv[sl] + hdv[sl]
            a = jnp.where(a > 0, a, 0.2 * a)
            e = jnp.exp(a - mv)
            ev[sl] = e
            pv[sl] = e * hv[sl]
        _fill_idx(idxd, dst_v)
        for j in range(4):
            pltpu.sync_copy(ev.at[pl.ds(j * 128, 128)],
                            dacc.at[idxd.at[j]], add=True)
            pltpu.sync_copy(pv.at[pl.ds(j * 128, 128)],
                            nacc.at[idxd.at[j]], add=True)

    plsc.subcore_barrier()
    _flush1(nacc, num_out, c * NPAD)
    _flush1(dacc, den_out, c * NPAD)


def _sc_kernel(body, out_shapes, scratch):
    mesh = plsc.VectorSubcoreMesh(**_MESH)
    return pl.kernel(body, out_type=out_shapes, mesh=mesh,
                     scratch_types=scratch,
                     compiler_params=pltpu.CompilerParams(
                         use_tc_tiling_on_sc=False))


def _sc_deg(dst, w):
    return _sc_kernel(
        _deg_body,
        [jax.ShapeDtypeStruct((2 * NPAD,), f32)],
        [pltpu.VMEM((BLK,), i32), pltpu.VMEM((BLK,), f32),
         pltpu.VMEM((4, 128), i32), pltpu.VMEM((ZR, 16), f32),
         pltpu.VMEM((ZR,), f32), pltpu.VMEM_SHARED((NPAD,), f32),
         pltpu.SemaphoreType.DMA],
    )(dst, w)[0]


def _sc_norm(src, dst, w, dinv):
    return _sc_kernel(
        _norm_body,
        [jax.ShapeDtypeStruct((EPAD,), f32)],
        [pltpu.VMEM((BLK,), i32), pltpu.VMEM((BLK,), i32),
         pltpu.VMEM((BLK,), f32), pltpu.VMEM((BLK,), f32),
         pltpu.VMEM((BLK,), f32), pltpu.VMEM((BLK,), f32),
         pltpu.SemaphoreType.DMA],
    )(src, dst, w, dinv)[0]


def _sc_gat(hlo, hhi, hs, hd, mvec, src, dst2):
    outs = _sc_kernel(
        _gat_body,
        [jax.ShapeDtypeStruct((4 * NPAD, 16), f32),
         jax.ShapeDtypeStruct((2 * NPAD,), f32),
         jax.ShapeDtypeStruct((EPAD,), f32)],
        [pltpu.VMEM((GB,), i32), pltpu.VMEM((GB,), i32),
         pltpu.VMEM((GB // 128, 128), i32),
         pltpu.VMEM((GB,), f32), pltpu.VMEM((GB,), f32),
         pltpu.VMEM((GB,), f32), pltpu.VMEM((GB, 16), f32),
         pltpu.VMEM((ZR, 16), f32),
         pltpu.VMEM((ZR,), f32), pltpu.VMEM((16,), f32),
         pltpu.VMEM_SHARED((NPAD, 16), f32),
         pltpu.VMEM_SHARED((NPAD,), f32),
         pltpu.SemaphoreType.DMA, pltpu.SemaphoreType.DMA,
         pltpu.SemaphoreType.DMA, pltpu.SemaphoreType.DMA,
         pltpu.SemaphoreType.DMA],
    )(hlo, hhi, hs, hd, mvec, src, dst2)
    return outs[0], outs[1]


def _sc_gcn(tabs, src, dst2, norm):
    return _sc_kernel(
        _gcn_body,
        [jax.ShapeDtypeStruct((10 * NPAD, 16), f32)],
        [pltpu.VMEM((GB,), i32), pltpu.VMEM((GB // 128, 128), i32),
         pltpu.VMEM((GB,), f32), pltpu.VMEM((GB, 16), f32),
         pltpu.VMEM((ZR, 16), f32),
         pltpu.VMEM_SHARED((NPAD, 16), f32),
         pltpu.SemaphoreType.DMA, pltpu.SemaphoreType.DMA,
         pltpu.SemaphoreType.DMA],
    )(*tabs, src, dst2, norm)[0]


def _sc_gat1(hs, hd, h, mvec, src, dst):
    outs = _sc_kernel(
        _gat1_body,
        [jax.ShapeDtypeStruct((2 * NPAD,), f32),
         jax.ShapeDtypeStruct((2 * NPAD,), f32)],
        [pltpu.VMEM((BLK,), i32), pltpu.VMEM((BLK,), i32),
         pltpu.VMEM((BLK,), f32), pltpu.VMEM((BLK,), f32),
         pltpu.VMEM((BLK,), f32), pltpu.VMEM((BLK,), f32),
         pltpu.VMEM((BLK,), f32), pltpu.VMEM((4, 128), i32),
         pltpu.VMEM((ZR,), f32), pltpu.VMEM_SHARED((NPAD,), f32),
         pltpu.VMEM_SHARED((NPAD,), f32), pltpu.VMEM((16,), f32),
         pltpu.SemaphoreType.DMA],
    )(hs, hd, h, mvec, src, dst)
    return outs[0], outs[1]


def _row_spec(d):
    return pl.BlockSpec((BR, d), lambda r: (r, 0))


def _full_spec(shape):
    nd = len(shape)
    return pl.BlockSpec(shape, lambda r: (0,) * nd)


def _lead_spec(lead, d):
    return pl.BlockSpec((lead, BR, d), lambda r: (0, r, 0))


def _tc_call(body, in_specs, out_shapes, out_specs):
    return pl.pallas_call(
        body, grid=(NG,), in_specs=in_specs,
        out_shape=[jax.ShapeDtypeStruct(s, f32) for s in out_shapes],
        out_specs=out_specs)


def _dgT(a, b):
    return lax.dot_general(a, b, (((1,), (1,)), ((), ())),
                           preferred_element_type=f32)


def _col_spec(d):
    return pl.BlockSpec((d, BR), lambda r: (0, r))


def _cvec_spec(lead):
    return pl.BlockSpec((lead, BR), lambda r: (0, r))


_VROW = pl.BlockSpec((1, BR), lambda r: (0, r))


def _tca_body(o0t_ref, w_ref, as_ref, ad_ref, hlo_ref, hhi_ref,
              hs_ref, hd_ref):
    h = lax.dot_general(o0t_ref[...], w_ref[...],
                        (((0,), (0,)), ((), ())),
                        preferred_element_type=f32)
    hlo_ref[...] = h[:, :16]
    hhi_ref[...] = h[:, 16:32]
    hs_ref[...] = _dgT(as_ref[...], h)
    hd_ref[...] = _dgT(ad_ref[...], h)


def _tca(o0t, W, a_s, a_d, k):
    return _tc_call(
        _tca_body,
        [_col_spec(k), _full_spec((k, HID)), _full_spec((1, HID)),
         _full_spec((1, HID))],
        [(NPAD, 16), (NPAD, 16), (1, NPAD), (1, NPAD)],
        [_row_spec(16), _row_spec(16), _VROW, _VROW])(
            o0t, W, a_s.reshape(1, HID), a_d.reshape(1, HID))


def _dinv_body(deg_ref, dinv_ref):
    d = deg_ref[...]
    dt = d[0:1] + d[1:2]
    dinv_ref[...] = jnp.where(dt > 0, lax.rsqrt(dt), 0.0)


def _dinv(deg2):
    return _tc_call(
        _dinv_body, [_cvec_spec(2)], [(1, NPAD)],
        [_VROW])(deg2.reshape(2, NPAD))[0]


def _tcb_body(num_ref, den_ref, b_ref, w_ref, as_ref, ad_ref, eye_ref,
              o1_ref, hlo_ref, hhi_ref, hs_ref, hd_ref):
    n = num_ref[...]
    dn = den_ref[...]
    numv = jnp.concatenate([n[0] + n[2], n[1] + n[3]], axis=1)
    dsum = dn[0:1] + dn[1:2]
    rcp = 1.0 / (dsum + 1e-16)
    rcp_col = lax.dot_general(eye_ref[...], rcp,
                              (((1,), (1,)), ((), ())),
                              preferred_element_type=f32)
    o1 = numv * rcp_col + b_ref[...]
    o1_ref[...] = o1
    h = jnp.dot(o1, w_ref[...], preferred_element_type=f32)
    hlo_ref[...] = h[:, :16]
    hhi_ref[...] = h[:, 16:32]
    hs_ref[...] = _dgT(as_ref[...], h)
    hd_ref[...] = _dgT(ad_ref[...], h)


def _tcb(num, den, b, W, a_s, a_d, eye):
    return _tc_call(
        _tcb_body,
        [_lead_spec(4, 16), _cvec_spec(2), _full_spec((1, HID)),
         _full_spec((HID, HID)), _full_spec((1, HID)),
         _full_spec((1, HID)), _full_spec((BR, BR))],
        [(NPAD, HID), (NPAD, 16), (NPAD, 16), (1, NPAD), (1, NPAD)],
        [_row_spec(HID), _row_spec(16), _row_spec(16), _VROW, _VROW])(
            num.reshape(4, NPAD, 16), den.reshape(2, NPAD),
            b.reshape(1, HID), W, a_s.reshape(1, HID),
            a_d.reshape(1, HID), eye)


def _tcc_body(num_ref, den_ref, b_ref, o0t_ref, o1_ref, wg0_ref,
              wg1_ref, wg2_ref, eye_ref, ht_ref):
    n = num_ref[...]
    dn = den_ref[...]
    numv = jnp.concatenate([n[0] + n[2], n[1] + n[3]], axis=1)
    rcp = 1.0 / (dn[0:1] + dn[1:2] + 1e-16)
    rcp_col = lax.dot_general(eye_ref[...], rcp,
                              (((1,), (1,)), ((), ())),
                              preferred_element_type=f32)
    o2 = numv * rcp_col + b_ref[...]
    ht = (lax.dot_general(o0t_ref[...], wg0_ref[...],
                          (((0,), (0,)), ((), ())),
                          preferred_element_type=f32)
          + jnp.dot(o1_ref[...], wg1_ref[...], preferred_element_type=f32)
          + jnp.dot(o2, wg2_ref[...], preferred_element_type=f32))
    ht_ref[...] = jnp.concatenate([ht, jnp.zeros((BR, 4), f32)], axis=1)


def _tcc(num, den, b, o0t, o1, Wg, eye):
    return _tc_call(
        _tcc_body,
        [_lead_spec(4, 16), _cvec_spec(2), _full_spec((1, HID)),
         _col_spec(T_PAST), _row_spec(HID), _full_spec((T_PAST, TEMP)),
         _full_spec((HID, TEMP)), _full_spec((HID, TEMP)),
         _full_spec((BR, BR))],
        [(NPAD, 80)],
        [_row_spec(80)])(
            num.reshape(4, NPAD, 16), den.reshape(2, NPAD),
            b.reshape(1, HID), o0t, o1, Wg[:T_PAST], Wg[T_PAST:44],
            Wg[44:], eye)


def _tcd_body(acc_ref, bg_ref, w3_ref, as_ref, ad_ref,
              h3_ref, hs_ref, hd_ref):
    a = acc_ref[...]
    g = jnp.concatenate([a[p] + a[5 + p] for p in range(5)], axis=1)
    xt = g[:, :TEMP] + bg_ref[...]
    h3 = _dgT(w3_ref[...], xt)
    h3_ref[...] = h3
    hs_ref[...] = h3 * as_ref[0, 0]
    hd_ref[...] = h3 * ad_ref[0, 0]


def _tcd(accg, bg, W3, a_s3, a_d3):
    return _tc_call(
        _tcd_body,
        [_lead_spec(10, 16), _full_spec((1, TEMP)),
         _full_spec((1, TEMP)), _full_spec((1, 1)), _full_spec((1, 1))],
        [(1, NPAD)] * 3,
        [_VROW] * 3)(
            accg.reshape(10, NPAD, 16), bg.reshape(1, TEMP),
            W3.reshape(1, TEMP), a_s3.reshape(1, 1), a_d3.reshape(1, 1))


def _tce_body(num_ref, den_ref, b_ref, o0t_ref, yp_ref, o0n_ref):
    n = num_ref[...]
    dn = den_ref[...]
    yp = (n[0:1] + n[1:2]) / (dn[0:1] + dn[1:2] + 1e-16) + b_ref[0, 0]
    yp = jnp.clip(yp, 0.0, 90.0)
    yp_ref[...] = yp
    o0n_ref[...] = jnp.concatenate([o0t_ref[...][1:], yp], axis=0)


def _tce(num, den, b3, o0t):
    return _tc_call(
        _tce_body,
        [_cvec_spec(2), _cvec_spec(2), _full_spec((1, 1)),
         _col_spec(T_PAST)],
        [(1, NPAD), (T_PAST, NPAD)],
        [_VROW, _col_spec(T_PAST)])(
            num.reshape(2, NPAD), den.reshape(2, NPAD),
            b3.reshape(1, 1), o0t)


def _leaky(v):
    return jnp.where(v > 0, v, 0.2 * v)


def _mvec(hs, hd):
    m = _leaky(jnp.max(hs.reshape(NPAD)[:N]) + jnp.max(hd.reshape(NPAD)[:N]))
    return jnp.full((16,), m, f32)


def kernel(x, edge_index, edge_attr, y, W0, a_src0, a_dst0, b0,
           W1, a_src1, a_dst1, b1, Wg, bg, W3, a_src3, a_dst3, b3):
    t_future = y.shape[1]
    loop = jnp.arange(N, dtype=edge_index.dtype)
    padn = EPAD - EE
    src = jnp.concatenate(
        [edge_index[0], loop, jnp.zeros((padn,), edge_index.dtype)])
    dst = jnp.concatenate(
        [edge_index[1], loop, jnp.full((padn,), N, edge_index.dtype)])
    w = jnp.concatenate(
        [edge_attr, jnp.ones((N,), f32), jnp.zeros((padn,), f32)])

    dst2 = dst.reshape(EPAD // 128, 128)
    deg2 = _sc_deg(dst, w)
    dinv = _dinv(deg2)
    norm = _sc_norm(src, dst, w, dinv.reshape(NPAD))

    o0t = jnp.concatenate(
        [x, jnp.zeros((NPAD - N, T_PAST), f32)], axis=0).T
    eye = jnp.eye(BR, dtype=f32)
    preds = []
    for _step in range(t_future):
        hlo, hhi, hs0, hd0 = _tca(o0t, W0, a_src0, a_dst0, T_PAST)
        num0, den0 = _sc_gat(hlo, hhi, hs0.reshape(NPAD),
                             hd0.reshape(NPAD), _mvec(hs0, hd0),
                             src, dst2)
        o1, h1lo, h1hi, hs1, hd1 = _tcb(num0, den0, b0, W1,
                                        a_src1, a_dst1, eye)
        num1, den1 = _sc_gat(h1lo, h1hi, hs1.reshape(NPAD),
                             hd1.reshape(NPAD), _mvec(hs1, hd1),
                             src, dst2)
        ht80 = _tcc(num1, den1, b1, o0t, o1, Wg, eye)[0]
        tabs = [ht80[:, 16 * p:16 * p + 16] for p in range(5)]
        accg = _sc_gcn(tabs, src, dst2, norm)
        h3, hs3, hd3 = _tcd(accg, bg, W3, a_src3, a_dst3)
        num3, den3 = _sc_gat1(hs3.reshape(NPAD), hd3.reshape(NPAD),
                              h3.reshape(NPAD), _mvec(hs3, hd3), src, dst)
        yp, o0t = _tce(num3, den3, b3, o0t)
        preds.append(yp.reshape(NPAD, 1)[:N])

    return jnp.concatenate(preds, axis=1)


# speedup vs baseline: 40.5116x; 1.0712x over previous
import functools

import jax
import jax.numpy as jnp
from jax import lax
from jax.experimental import pallas as pl
from jax.experimental.pallas import tpu as pltpu
from jax.experimental.pallas import tpu_sc as plsc

N = 100000
T_PAST = 12
HID = 32
TEMP = T_PAST + 2 * HID

BR = 512
NPAD = 196 * BR
NG = NPAD // BR
RPT = NPAD // 16
ZR = 64

E = 1600000
EE = E + N
BLK = 512
EPT = 53248
EPAD = 32 * EPT
NBLK = EPT // BLK
HALF_E = EPAD // 2

_MESH = dict(core_axis_name="c", subcore_axis_name="s",
             num_cores=2, num_subcores=16)

f32 = jnp.float32
i32 = jnp.int32


def _edge_base(i):
    c = lax.axis_index("c")
    s = lax.axis_index("s")
    return c * HALF_E + s * EPT + i * BLK


def _fill_idx(idxd, dst_v):
    for k in range(32):
        idxd[k // 8, pl.ds((k % 8) * 16, 16)] = dst_v[pl.ds(k * 16, 16)]


def _zero_rows(zbuf, acc):
    s = lax.axis_index("s")

    @pl.loop(0, RPT // ZR)
    def _(i):
        pltpu.sync_copy(zbuf, acc.at[pl.ds(s * RPT + i * ZR, ZR), :])


def _zero_rows1(zd, acc1):
    s = lax.axis_index("s")

    @pl.loop(0, RPT // ZR)
    def _(i):
        pltpu.sync_copy(zd, acc1.at[pl.ds(s * RPT + i * ZR, ZR)])


def _init_zbufs(zbuf, zd):
    zro = jnp.zeros((16,), f32)

    @pl.loop(0, ZR)
    def _(i):
        zbuf[i] = zro

    @pl.loop(0, ZR // 16)
    def _(i):
        zd[pl.ds(i * 16, 16)] = zro


def _flush(acc, out, row0):
    s = lax.axis_index("s")
    pltpu.sync_copy(acc.at[pl.ds(s * RPT, RPT), :],
                    out.at[pl.ds(row0 + s * RPT, RPT), :])


def _flush1(acc1, out1, row0):
    s = lax.axis_index("s")
    pltpu.sync_copy(acc1.at[pl.ds(s * RPT, RPT)],
                    out1.at[pl.ds(row0 + s * RPT, RPT)])


def _deg_body(dst_h, w_h, deg_out, dst_v, w_v, idxd, zbuf, zd, acc1, sem):
    c = lax.axis_index("c")
    _init_zbufs(zbuf, zd)
    _zero_rows1(zd, acc1)
    plsc.subcore_barrier()

    @pl.loop(0, NBLK)
    def _(i):
        be = _edge_base(i)
        pltpu.sync_copy(dst_h.at[pl.ds(be, BLK)], dst_v)
        pltpu.sync_copy(w_h.at[pl.ds(be, BLK)], w_v)
        _fill_idx(idxd, dst_v)
        for j in range(4):
            pltpu.sync_copy(w_v.at[pl.ds(j * 128, 128)],
                            acc1.at[idxd.at[j]], add=True)

    plsc.subcore_barrier()
    _flush1(acc1, deg_out, c * NPAD)


def _norm_body(src_h, dst_h, w_h, dinv_h, norm_out,
               src_v, dst_v, w_v, dv, dd, nv, sem):
    @pl.loop(0, NBLK)
    def _(i):
        be = _edge_base(i)
        pltpu.sync_copy(src_h.at[pl.ds(be, BLK)], src_v)
        pltpu.sync_copy(dst_h.at[pl.ds(be, BLK)], dst_v)
        pltpu.sync_copy(w_h.at[pl.ds(be, BLK)], w_v)
        pltpu.async_copy(dinv_h.at[src_v], dv, sem).wait()
        pltpu.async_copy(dinv_h.at[dst_v], dd, sem).wait()
        for k in range(32):
            sl = pl.ds(k * 16, 16)
            nv[sl] = dv[sl] * w_v[sl] * dd[sl]
        pltpu.sync_copy(nv, norm_out.at[pl.ds(be, BLK)])


GB = 1024


def _batch_base(i):
    c = lax.axis_index("c")
    s = lax.axis_index("s")
    return c * HALF_E + s * EPT + i * GB


def _gat_body(hlo, hhi, hs_h, hd_h, m_h, src_h, dst2_h,
              num_out, den_out, ebuf,
              src_v, dst1_v, dst2_v, hsv, hdv, ev, rows_v, zbuf, zd, m_v,
              acc, dacc, semL, semR, semA, semB, semS):
    c = lax.axis_index("c")
    _init_zbufs(zbuf, zd)
    pltpu.sync_copy(m_h, m_v)
    mv = m_v[...]

    for f, tab in enumerate((hlo, hhi)):
        _zero_rows(zbuf, acc)
        if f == 0:
            _zero_rows1(zd, dacc)
        plsc.subcore_barrier()

        @pl.loop(0, NBLK // 2)
        def _(i):
            be = _batch_base(i)
            br = be // 128
            cs = pltpu.async_copy(src_h.at[pl.ds(be, GB)], src_v, semL)
            cd = pltpu.async_copy(dst2_h.at[pl.ds(br, GB // 128), :],
                                  dst2_v, semL)
            cs.wait()
            cd.wait()
            gr = pltpu.async_copy(tab.at[src_v], rows_v, semR)
            if f == 0:
                for j in range(GB // 128):
                    for m in range(8):
                        dst1_v[pl.ds(j * 128 + m * 16, 16)] = (
                            dst2_v[j, pl.ds(m * 16, 16)])
                ga = pltpu.async_copy(hs_h.at[src_v], hsv, semA)
                gb = pltpu.async_copy(hd_h.at[dst1_v], hdv, semB)
                ga.wait()
                gb.wait()
                for k in range(GB // 16):
                    sl = pl.ds(k * 16, 16)
                    a = hsv[sl] + hdv[sl]
                    a = jnp.where(a > 0, a, 0.2 * a)
                    ev[sl] = jnp.exp(a - mv)
                ce = pltpu.async_copy(ev, ebuf.at[pl.ds(be, GB)], semL)
            else:
                pltpu.async_copy(ebuf.at[pl.ds(be, GB)], ev, semL).wait()
            gr.wait()

            @pl.loop(0, GB // 16)
            def _(k):
                e16 = ev[pl.ds(k * 16, 16)]
                for u in range(16):
                    t = k * 16 + u
                    rows_v[t] = rows_v[t] * e16[u]

            sc = []
            for j in range(GB // 128):
                sc.append(pltpu.async_copy(
                    rows_v.at[pl.ds(j * 128, 128), :],
                    acc.at[dst2_v.at[j]], semS, add=True))
                if f == 0:
                    sc.append(pltpu.async_copy(
                        ev.at[pl.ds(j * 128, 128)],
                        dacc.at[dst2_v.at[j]], semS, add=True))
            for d in sc:
                d.wait()
            if f == 0:
                ce.wait()

        plsc.subcore_barrier()
        _flush(acc, num_out, (c * 2 + f) * NPAD)
        if f == 0:
            _flush1(dacc, den_out, c * NPAD)
        plsc.subcore_barrier()


def _gcn_body(t0, t1, t2, t3, t4, src_h, dst2_h, norm_h, acc_out,
              src_v, dst2_v, nv, rows_v, zbuf, acc,
              semL, semR, semS):
    c = lax.axis_index("c")
    zro = jnp.zeros((16,), f32)

    @pl.loop(0, ZR)
    def _(i):
        zbuf[i] = zro

    for p, tab in enumerate((t0, t1, t2, t3, t4)):
        _zero_rows(zbuf, acc)
        plsc.subcore_barrier()

        @pl.loop(0, NBLK // 2)
        def _(i):
            be = _batch_base(i)
            br = be // 128
            cs = pltpu.async_copy(src_h.at[pl.ds(be, GB)], src_v, semL)
            cd = pltpu.async_copy(dst2_h.at[pl.ds(br, GB // 128), :],
                                  dst2_v, semL)
            cn = pltpu.async_copy(norm_h.at[pl.ds(be, GB)], nv, semL)
            cs.wait()
            cd.wait()
            gr = pltpu.async_copy(tab.at[src_v], rows_v, semR)
            cn.wait()
            gr.wait()

            @pl.loop(0, GB // 16)
            def _(k):
                e16 = nv[pl.ds(k * 16, 16)]
                for u in range(16):
                    t = k * 16 + u
                    rows_v[t] = rows_v[t] * e16[u]

            sc = []
            for j in range(GB // 128):
                sc.append(pltpu.async_copy(
                    rows_v.at[pl.ds(j * 128, 128), :],
                    acc.at[dst2_v.at[j]], semS, add=True))
            for d in sc:
                d.wait()

        plsc.subcore_barrier()
        _flush(acc, acc_out, (c * 5 + p) * NPAD)
        plsc.subcore_barrier()


def _gat1_body(hs_h, hd_h, h_h, m_h, src_h, dst_h, num_out, den_out,
               src_v, dst_v, hsv, hdv, hv, ev, pv, idxd, zd, nacc, dacc,
               m_v, sem):
    c = lax.axis_index("c")
    zro = jnp.zeros((16,), f32)

    @pl.loop(0, ZR // 16)
    def _(i):
        zd[pl.ds(i * 16, 16)] = zro

    pltpu.sync_copy(m_h, m_v)
    mv = m_v[...]
    _zero_rows1(zd, nacc)
    _zero_rows1(zd, dacc)
    plsc.subcore_barrier()

    @pl.loop(0, NBLK)
    def _(i):
        be = _edge_base(i)
        pltpu.sync_copy(src_h.at[pl.ds(be, BLK)], src_v)
        pltpu.sync_copy(dst_h.at[pl.ds(be, BLK)], dst_v)
        pltpu.async_copy(hs_h.at[src_v], hsv, sem).wait()
        pltpu.async_copy(hd_h.at[dst_v], hdv, sem).wait()
        pltpu.async_copy(h_h.at[src_v], hv, sem).wait()
        for k in range(32):
            sl = pl.ds(k * 16, 16)
            a = hsv[sl] + hdv[sl]
            a = jnp.where(a > 0, a, 0.2 * a)
            e = jnp.exp(a - mv)
            ev[sl] = e
            pv[sl] = e * hv[sl]
        _fill_idx(idxd, dst_v)
        for j in range(4):
            pltpu.sync_copy(ev.at[pl.ds(j * 128, 128)],
                            dacc.at[idxd.at[j]], add=True)
            pltpu.sync_copy(pv.at[pl.ds(j * 128, 128)],
                            nacc.at[idxd.at[j]], add=True)

    plsc.subcore_barrier()
    _flush1(nacc, num_out, c * NPAD)
    _flush1(dacc, den_out, c * NPAD)


def _sc_kernel(body, out_shapes, scratch):
    mesh = plsc.VectorSubcoreMesh(**_MESH)
    return pl.kernel(body, out_type=out_shapes, mesh=mesh,
                     scratch_types=scratch,
                     compiler_params=pltpu.CompilerParams(
                         use_tc_tiling_on_sc=False))


def _sc_deg(dst, w):
    return _sc_kernel(
        _deg_body,
        [jax.ShapeDtypeStruct((2 * NPAD,), f32)],
        [pltpu.VMEM((BLK,), i32), pltpu.VMEM((BLK,), f32),
         pltpu.VMEM((4, 128), i32), pltpu.VMEM((ZR, 16), f32),
         pltpu.VMEM((ZR,), f32), pltpu.VMEM_SHARED((NPAD,), f32),
         pltpu.SemaphoreType.DMA],
    )(dst, w)[0]


def _sc_norm(src, dst, w, dinv):
    return _sc_kernel(
        _norm_body,
        [jax.ShapeDtypeStruct((EPAD,), f32)],
        [pltpu.VMEM((BLK,), i32), pltpu.VMEM((BLK,), i32),
         pltpu.VMEM((BLK,), f32), pltpu.VMEM((BLK,), f32),
         pltpu.VMEM((BLK,), f32), pltpu.VMEM((BLK,), f32),
         pltpu.SemaphoreType.DMA],
    )(src, dst, w, dinv)[0]


def _sc_gat(hlo, hhi, hs, hd, mvec, src, dst2):
    outs = _sc_kernel(
        _gat_body,
        [jax.ShapeDtypeStruct((4 * NPAD, 16), f32),
         jax.ShapeDtypeStruct((2 * NPAD,), f32),
         jax.ShapeDtypeStruct((EPAD,), f32)],
        [pltpu.VMEM((GB,), i32), pltpu.VMEM((GB,), i32),
         pltpu.VMEM((GB // 128, 128), i32),
         pltpu.VMEM((GB,), f32), pltpu.VMEM((GB,), f32),
         pltpu.VMEM((GB,), f32), pltpu.VMEM((GB, 16), f32),
         pltpu.VMEM((ZR, 16), f32),
         pltpu.VMEM((ZR,), f32), pltpu.VMEM((16,), f32),
         pltpu.VMEM_SHARED((NPAD, 16), f32),
         pltpu.VMEM_SHARED((NPAD,), f32),
         pltpu.SemaphoreType.DMA, pltpu.SemaphoreType.DMA,
         pltpu.SemaphoreType.DMA, pltpu.SemaphoreType.DMA,
         pltpu.SemaphoreType.DMA],
    )(hlo, hhi, hs, hd, mvec, src, dst2)
    return outs[0], outs[1]


def _sc_gcn(tabs, src, dst2, norm):
    return _sc_kernel(
        _gcn_body,
        [jax.ShapeDtypeStruct((10 * NPAD, 16), f32)],
        [pltpu.VMEM((GB,), i32), pltpu.VMEM((GB // 128, 128), i32),
         pltpu.VMEM((GB,), f32), pltpu.VMEM((GB, 16), f32),
         pltpu.VMEM((ZR, 16), f32),
         pltpu.VMEM_SHARED((NPAD, 16), f32),
         pltpu.SemaphoreType.DMA, pltpu.SemaphoreType.DMA,
         pltpu.SemaphoreType.DMA],
    )(*tabs, src, dst2, norm)[0]


def _sc_gat1(hs, hd, h, mvec, src, dst):
    outs = _sc_kernel(
        _gat1_body,
        [jax.ShapeDtypeStruct((2 * NPAD,), f32),
         jax.ShapeDtypeStruct((2 * NPAD,), f32)],
        [pltpu.VMEM((BLK,), i32), pltpu.VMEM((BLK,), i32),
         pltpu.VMEM((BLK,), f32), pltpu.VMEM((BLK,), f32),
         pltpu.VMEM((BLK,), f32), pltpu.VMEM((BLK,), f32),
         pltpu.VMEM((BLK,), f32), pltpu.VMEM((4, 128), i32),
         pltpu.VMEM((ZR,), f32), pltpu.VMEM_SHARED((NPAD,), f32),
         pltpu.VMEM_SHARED((NPAD,), f32), pltpu.VMEM((16,), f32),
         pltpu.SemaphoreType.DMA],
    )(hs, hd, h, mvec, src, dst)
    return outs[0], outs[1]


def _row_spec(d):
    return pl.BlockSpec((BR, d), lambda r: (r, 0))


def _full_spec(shape):
    nd = len(shape)
    return pl.BlockSpec(shape, lambda r: (0,) * nd)


def _lead_spec(lead, d):
    return pl.BlockSpec((lead, BR, d), lambda r: (0, r, 0))


def _tc_call(body, in_specs, out_shapes, out_specs):
    return pl.pallas_call(
        body, grid=(NG,), in_specs=in_specs,
        out_shape=[jax.ShapeDtypeStruct(s, f32) for s in out_shapes],
        out_specs=out_specs)


def _dgT(a, b):
    return lax.dot_general(a, b, (((1,), (1,)), ((), ())),
                           preferred_element_type=f32)


def _col_spec(d):
    return pl.BlockSpec((d, BR), lambda r: (0, r))


def _cvec_spec(lead):
    return pl.BlockSpec((lead, BR), lambda r: (0, r))


_VROW = pl.BlockSpec((1, BR), lambda r: (0, r))


def _tca_body(o0t_ref, w_ref, as_ref, ad_ref, hlo_ref, hhi_ref,
              hs_ref, hd_ref):
    h = lax.dot_general(o0t_ref[...], w_ref[...],
                        (((0,), (0,)), ((), ())),
                        preferred_element_type=f32)
    hlo_ref[...] = h[:, :16]
    hhi_ref[...] = h[:, 16:32]
    hs_ref[...] = _dgT(as_ref[...], h)
    hd_ref[...] = _dgT(ad_ref[...], h)


def _tca(o0t, W, a_s, a_d, k):
    return _tc_call(
        _tca_body,
        [_col_spec(k), _full_spec((k, HID)), _full_spec((1, HID)),
         _full_spec((1, HID))],
        [(NPAD, 16), (NPAD, 16), (1, NPAD), (1, NPAD)],
        [_row_spec(16), _row_spec(16), _VROW, _VROW])(
            o0t, W, a_s.reshape(1, HID), a_d.reshape(1, HID))


def _dinv_body(deg_ref, dinv_ref):
    d = deg_ref[...]
    dt = d[0:1] + d[1:2]
    dinv_ref[...] = jnp.where(dt > 0, lax.rsqrt(dt), 0.0)


def _dinv(deg2):
    return _tc_call(
        _dinv_body, [_cvec_spec(2)], [(1, NPAD)],
        [_VROW])(deg2.reshape(2, NPAD))[0]


def _tcb_body(num_ref, den_ref, b_ref, w_ref, as_ref, ad_ref, eye_ref,
              o1_ref, hlo_ref, hhi_ref, hs_ref, hd_ref):
    n = num_ref[...]
    dn = den_ref[...]
    numv = jnp.concatenate([n[0] + n[2], n[1] + n[3]], axis=1)
    dsum = dn[0:1] + dn[1:2]
    rcp = 1.0 / (dsum + 1e-16)
    rcp_col = lax.dot_general(eye_ref[...], rcp,
                              (((1,), (1,)), ((), ())),
                              preferred_element_type=f32)
    o1 = numv * rcp_col + b_ref[...]
    o1_ref[...] = o1
    h = jnp.dot(o1, w_ref[...], preferred_element_type=f32)
    hlo_ref[...] = h[:, :16]
    hhi_ref[...] = h[:, 16:32]
    hs_ref[...] = _dgT(as_ref[...], h)
    hd_ref[...] = _dgT(ad_ref[...], h)


def _tcb(num, den, b, W, a_s, a_d, eye):
    return _tc_call(
        _tcb_body,
        [_lead_spec(4, 16), _cvec_spec(2), _full_spec((1, HID)),
         _full_spec((HID, HID)), _full_spec((1, HID)),
         _full_spec((1, HID)), _full_spec((BR, BR))],
        [(NPAD, HID), (NPAD, 16), (NPAD, 16), (1, NPAD), (1, NPAD)],
        [_row_spec(HID), _row_spec(16), _row_spec(16), _VROW, _VROW])(
            num.reshape(4, NPAD, 16), den.reshape(2, NPAD),
            b.reshape(1, HID), W, a_s.reshape(1, HID),
            a_d.reshape(1, HID), eye)


def _tcc_body(num_ref, den_ref, b_ref, o0t_ref, o1_ref, wg0_ref,
              wg1_ref, wg2_ref, eye_ref, ht_ref):
    n = num_ref[...]
    dn = den_ref[...]
    numv = jnp.concatenate([n[0] + n[2], n[1] + n[3]], axis=1)
    rcp = 1.0 / (dn[0:1] + dn[1:2] + 1e-16)
    rcp_col = lax.dot_general(eye_ref[...], rcp,
                              (((1,), (1,)), ((), ())),
                              preferred_element_type=f32)
    o2 = numv * rcp_col + b_ref[...]
    ht = (lax.dot_general(o0t_ref[...], wg0_ref[...],
                          (((0,), (0,)), ((), ())),
                          preferred_element_type=f32)
          + jnp.dot(o1_ref[...], wg1_ref[...], preferred_element_type=f32)
          + jnp.dot(o2, wg2_ref[...], preferred_element_type=f32))
    ht_ref[...] = jnp.concatenate([ht, jnp.zeros((BR, 4), f32)], axis=1)


def _tcc(num, den, b, o0t, o1, Wg, eye):
    return _tc_call(
        _tcc_body,
        [_lead_spec(4, 16), _cvec_spec(2), _full_spec((1, HID)),
         _col_spec(T_PAST), _row_spec(HID), _full_spec((T_PAST, TEMP)),
         _full_spec((HID, TEMP)), _full_spec((HID, TEMP)),
         _full_spec((BR, BR))],
        [(NPAD, 80)],
        [_row_spec(80)])(
            num.reshape(4, NPAD, 16), den.reshape(2, NPAD),
            b.reshape(1, HID), o0t, o1, Wg[:T_PAST], Wg[T_PAST:44],
            Wg[44:], eye)


def _tcd_body(acc_ref, wk_ref, b_ref, out_ref):
    a = acc_ref[...]
    wk = wk_ref[...]
    tot = jnp.zeros((NPAD // 8 // NG, 128), f32) + b_ref[0, 0]
    for p in range(5):
        tot = tot + jnp.dot(a[p] + a[5 + p], wk[p],
                            preferred_element_type=f32)
    out_ref[...] = tot


def _tcd(accg, Wk3, bgw3):
    npk = NPAD // 8
    return _tc_call(
        _tcd_body,
        [pl.BlockSpec((10, npk // NG, 128), lambda r: (0, r, 0)),
         _full_spec((5, 128, 128)), _full_spec((1, 1))],
        [(npk, 128)],
        [pl.BlockSpec((npk // NG, 128), lambda r: (r, 0))])(
            accg.reshape(10, npk, 128), Wk3, bgw3.reshape(1, 1))


def _tce_body(num_ref, den_ref, b_ref, o0t_ref, yp_ref, o0n_ref):
    n = num_ref[...]
    dn = den_ref[...]
    yp = (n[0:1] + n[1:2]) / (dn[0:1] + dn[1:2] + 1e-16) + b_ref[0, 0]
    yp = jnp.clip(yp, 0.0, 90.0)
    yp_ref[...] = yp
    o0n_ref[...] = jnp.concatenate([o0t_ref[...][1:], yp], axis=0)


def _tce(num, den, b3, o0t):
    return _tc_call(
        _tce_body,
        [_cvec_spec(2), _cvec_spec(2), _full_spec((1, 1)),
         _col_spec(T_PAST)],
        [(1, NPAD), (T_PAST, NPAD)],
        [_VROW, _col_spec(T_PAST)])(
            num.reshape(2, NPAD), den.reshape(2, NPAD),
            b3.reshape(1, 1), o0t)


def _leaky(v):
    return jnp.where(v > 0, v, 0.2 * v)


def _mvec(hs, hd):
    m = _leaky(jnp.max(hs.reshape(NPAD)[:N]) + jnp.max(hd.reshape(NPAD)[:N]))
    return jnp.full((16,), m, f32)


def kernel(x, edge_index, edge_attr, y, W0, a_src0, a_dst0, b0,
           W1, a_src1, a_dst1, b1, Wg, bg, W3, a_src3, a_dst3, b3):
    t_future = y.shape[1]
    loop = jnp.arange(N, dtype=edge_index.dtype)
    padn = EPAD - EE
    src = jnp.concatenate(
        [edge_index[0], loop, jnp.zeros((padn,), edge_index.dtype)])
    dst = jnp.concatenate(
        [edge_index[1], loop, jnp.full((padn,), N, edge_index.dtype)])
    w = jnp.concatenate(
        [edge_attr, jnp.ones((N,), f32), jnp.zeros((padn,), f32)])

    dst2 = dst.reshape(EPAD // 128, 128)
    deg2 = _sc_deg(dst, w)
    dinv = _dinv(deg2)
    norm = _sc_norm(src, dst, w, dinv.reshape(NPAD))

    o0t = jnp.concatenate(
        [x, jnp.zeros((NPAD - N, T_PAST), f32)], axis=0).T
    eye = jnp.eye(BR, dtype=f32)
    w3pad = jnp.concatenate([W3[:, 0], jnp.zeros((4,), f32)])
    Wk3 = jnp.stack([
        jnp.kron(jnp.eye(8, dtype=f32),
                 w3pad[16 * p:16 * p + 16][:, None] * jnp.ones((1, 16), f32))
        for p in range(5)])
    bgw3 = jnp.dot(bg, W3[:, 0]).reshape(1, 1)
    preds = []
    for _step in range(t_future):
        hlo, hhi, hs0, hd0 = _tca(o0t, W0, a_src0, a_dst0, T_PAST)
        num0, den0 = _sc_gat(hlo, hhi, hs0.reshape(NPAD),
                             hd0.reshape(NPAD), _mvec(hs0, hd0),
                             src, dst2)
        o1, h1lo, h1hi, hs1, hd1 = _tcb(num0, den0, b0, W1,
                                        a_src1, a_dst1, eye)
        num1, den1 = _sc_gat(h1lo, h1hi, hs1.reshape(NPAD),
                             hd1.reshape(NPAD), _mvec(hs1, hd1),
                             src, dst2)
        ht80 = _tcc(num1, den1, b1, o0t, o1, Wg, eye)[0]
        tabs = [ht80[:, 16 * p:16 * p + 16] for p in range(5)]
        accg = _sc_gcn(tabs, src, dst2, norm)
        h3rep = _tcd(accg, Wk3, bgw3)[0]
        h3f = h3rep.reshape(NPAD, 16)[:, 0]
        hs3f = h3f * a_src3[0]
        hd3f = h3f * a_dst3[0]
        m3 = jnp.full((16,), _leaky(jnp.max(hs3f[:N]) + jnp.max(hd3f[:N])),
                      f32)
        num3, den3 = _sc_gat1(hs3f, hd3f, h3f, m3, src, dst)
        yp, o0t = _tce(num3, den3, b3, o0t)
        preds.append(yp.reshape(NPAD, 1)[:N])

    return jnp.concatenate(preds, axis=1)


# speedup vs baseline: 43.9075x; 1.0838x over previous
import functools

import jax
import jax.numpy as jnp
from jax import lax
from jax.experimental import pallas as pl
from jax.experimental.pallas import tpu as pltpu
from jax.experimental.pallas import tpu_sc as plsc

N = 100000
T_PAST = 12
HID = 32
TEMP = T_PAST + 2 * HID

BR = 512
NPAD = 196 * BR
NG = NPAD // BR
RPT = NPAD // 16
ZR = 64

E = 1600000
EE = E + N
BLK = 512
EPT = 53248
EPAD = 32 * EPT
NBLK = EPT // BLK
HALF_E = EPAD // 2

_MESH = dict(core_axis_name="c", subcore_axis_name="s",
             num_cores=2, num_subcores=16)

f32 = jnp.float32
i32 = jnp.int32


def _edge_base(i):
    c = lax.axis_index("c")
    s = lax.axis_index("s")
    return c * HALF_E + s * EPT + i * BLK


def _fill_idx(idxd, dst_v):
    for k in range(32):
        idxd[k // 8, pl.ds((k % 8) * 16, 16)] = dst_v[pl.ds(k * 16, 16)]


def _zero_rows(zbuf, acc):
    s = lax.axis_index("s")

    @pl.loop(0, RPT // ZR)
    def _(i):
        pltpu.sync_copy(zbuf, acc.at[pl.ds(s * RPT + i * ZR, ZR), :])


def _zero_rows1(zd, acc1):
    s = lax.axis_index("s")

    @pl.loop(0, RPT // ZR)
    def _(i):
        pltpu.sync_copy(zd, acc1.at[pl.ds(s * RPT + i * ZR, ZR)])


def _init_zbufs(zbuf, zd):
    zro = jnp.zeros((16,), f32)

    @pl.loop(0, ZR)
    def _(i):
        zbuf[i] = zro

    @pl.loop(0, ZR // 16)
    def _(i):
        zd[pl.ds(i * 16, 16)] = zro


def _flush(acc, out, row0):
    s = lax.axis_index("s")
    pltpu.sync_copy(acc.at[pl.ds(s * RPT, RPT), :],
                    out.at[pl.ds(row0 + s * RPT, RPT), :])


def _flush1(acc1, out1, row0):
    s = lax.axis_index("s")
    pltpu.sync_copy(acc1.at[pl.ds(s * RPT, RPT)],
                    out1.at[pl.ds(row0 + s * RPT, RPT)])


def _deg_body(dst2_h, w_h, deg_out, w_v, dst2_v, zd, acc1, semL, semS):
    c = lax.axis_index("c")
    zro = jnp.zeros((16,), f32)

    @pl.loop(0, ZR // 16)
    def _(i):
        zd[pl.ds(i * 16, 16)] = zro

    _zero_rows1(zd, acc1)
    plsc.subcore_barrier()

    @pl.loop(0, NBLK // 2)
    def _(i):
        be = _batch_base(i)
        cw = pltpu.async_copy(w_h.at[pl.ds(be, GB)], w_v, semL)
        cd = pltpu.async_copy(dst2_h.at[pl.ds(be // 128, GB // 128), :],
                              dst2_v, semL)
        cw.wait()
        cd.wait()
        sc = []
        for j in range(GB // 128):
            sc.append(pltpu.async_copy(
                w_v.at[pl.ds(j * 128, 128)],
                acc1.at[dst2_v.at[j]], semS, add=True))
        for d in sc:
            d.wait()

    plsc.subcore_barrier()
    _flush1(acc1, deg_out, c * NPAD)


def _norm_body(src_h, dst2_h, w_h, dinv_h, norm_out,
               src_v, dst1_v, dst2_v, w_v, dv, dd, nv,
               semL, semA, semB):
    @pl.loop(0, NBLK // 2)
    def _(i):
        be = _batch_base(i)
        cs = pltpu.async_copy(src_h.at[pl.ds(be, GB)], src_v, semL)
        cd = pltpu.async_copy(dst2_h.at[pl.ds(be // 128, GB // 128), :],
                              dst2_v, semL)
        cw = pltpu.async_copy(w_h.at[pl.ds(be, GB)], w_v, semL)
        cs.wait()
        cd.wait()
        cw.wait()
        for j in range(GB // 128):
            for m in range(8):
                dst1_v[pl.ds(j * 128 + m * 16, 16)] = (
                    dst2_v[j, pl.ds(m * 16, 16)])
        ga = pltpu.async_copy(dinv_h.at[src_v], dv, semA)
        gb = pltpu.async_copy(dinv_h.at[dst1_v], dd, semB)
        ga.wait()
        gb.wait()
        for k in range(GB // 16):
            sl = pl.ds(k * 16, 16)
            nv[sl] = dv[sl] * w_v[sl] * dd[sl]
        pltpu.sync_copy(nv, norm_out.at[pl.ds(be, GB)])

GB = 1024


def _batch_base(i):
    c = lax.axis_index("c")
    s = lax.axis_index("s")
    return c * HALF_E + s * EPT + i * GB


def _gat_body(hlo, hhi, hs_h, hd_h, m_h, src_h, dst2_h,
              num_out, den_out, ebuf,
              src_v, dst1_v, dst2_v, hsv, hdv, ev, rows_v, zbuf, zd, m_v,
              acc, dacc, semL, semR, semA, semB, semS):
    c = lax.axis_index("c")
    _init_zbufs(zbuf, zd)
    pltpu.sync_copy(m_h, m_v)
    mv = m_v[...]

    for f, tab in enumerate((hlo, hhi)):
        _zero_rows(zbuf, acc)
        if f == 0:
            _zero_rows1(zd, dacc)
        plsc.subcore_barrier()

        @pl.loop(0, NBLK // 2)
        def _(i):
            be = _batch_base(i)
            br = be // 128
            cs = pltpu.async_copy(src_h.at[pl.ds(be, GB)], src_v, semL)
            cd = pltpu.async_copy(dst2_h.at[pl.ds(br, GB // 128), :],
                                  dst2_v, semL)
            cs.wait()
            cd.wait()
            gr = pltpu.async_copy(tab.at[src_v], rows_v, semR)
            if f == 0:
                for j in range(GB // 128):
                    for m in range(8):
                        dst1_v[pl.ds(j * 128 + m * 16, 16)] = (
                            dst2_v[j, pl.ds(m * 16, 16)])
                ga = pltpu.async_copy(hs_h.at[src_v], hsv, semA)
                gb = pltpu.async_copy(hd_h.at[dst1_v], hdv, semB)
                ga.wait()
                gb.wait()
                for k in range(GB // 16):
                    sl = pl.ds(k * 16, 16)
                    a = hsv[sl] + hdv[sl]
                    a = jnp.where(a > 0, a, 0.2 * a)
                    ev[sl] = jnp.exp(a - mv)
                ce = pltpu.async_copy(ev, ebuf.at[pl.ds(be, GB)], semL)
            else:
                pltpu.async_copy(ebuf.at[pl.ds(be, GB)], ev, semL).wait()
            gr.wait()

            @pl.loop(0, GB // 16)
            def _(k):
                e16 = ev[pl.ds(k * 16, 16)]
                for u in range(16):
                    t = k * 16 + u
                    rows_v[t] = rows_v[t] * e16[u]

            sc = []
            for j in range(GB // 128):
                sc.append(pltpu.async_copy(
                    rows_v.at[pl.ds(j * 128, 128), :],
                    acc.at[dst2_v.at[j]], semS, add=True))
                if f == 0:
                    sc.append(pltpu.async_copy(
                        ev.at[pl.ds(j * 128, 128)],
                        dacc.at[dst2_v.at[j]], semS, add=True))
            for d in sc:
                d.wait()
            if f == 0:
                ce.wait()

        plsc.subcore_barrier()
        _flush(acc, num_out, (c * 2 + f) * NPAD)
        if f == 0:
            _flush1(dacc, den_out, c * NPAD)
        plsc.subcore_barrier()


def _gcn_body(t0, t1, t2, t3, t4, src_h, dst2_h, norm_h, acc_out,
              src_v, dst2_v, nv, rows_v, zbuf, acc,
              semL, semR, semS):
    c = lax.axis_index("c")
    zro = jnp.zeros((16,), f32)

    @pl.loop(0, ZR)
    def _(i):
        zbuf[i] = zro

    for p, tab in enumerate((t0, t1, t2, t3, t4)):
        _zero_rows(zbuf, acc)
        plsc.subcore_barrier()

        @pl.loop(0, NBLK // 2)
        def _(i):
            be = _batch_base(i)
            br = be // 128
            cs = pltpu.async_copy(src_h.at[pl.ds(be, GB)], src_v, semL)
            cd = pltpu.async_copy(dst2_h.at[pl.ds(br, GB // 128), :],
                                  dst2_v, semL)
            cn = pltpu.async_copy(norm_h.at[pl.ds(be, GB)], nv, semL)
            cs.wait()
            cd.wait()
            gr = pltpu.async_copy(tab.at[src_v], rows_v, semR)
            cn.wait()
            gr.wait()

            @pl.loop(0, GB // 16)
            def _(k):
                e16 = nv[pl.ds(k * 16, 16)]
                for u in range(16):
                    t = k * 16 + u
                    rows_v[t] = rows_v[t] * e16[u]

            sc = []
            for j in range(GB // 128):
                sc.append(pltpu.async_copy(
                    rows_v.at[pl.ds(j * 128, 128), :],
                    acc.at[dst2_v.at[j]], semS, add=True))
            for d in sc:
                d.wait()

        plsc.subcore_barrier()
        _flush(acc, acc_out, (c * 5 + p) * NPAD)
        plsc.subcore_barrier()


def _gat1_body(hs_h, hd_h, h_h, m_h, src_h, dst2_h, num_out, den_out,
               src_v, dst1_v, dst2_v, hsv, hdv, hv, ev, pv, zd,
               nacc, dacc, m_v, semL, semR, semA, semB, semS):
    c = lax.axis_index("c")
    zro = jnp.zeros((16,), f32)

    @pl.loop(0, ZR // 16)
    def _(i):
        zd[pl.ds(i * 16, 16)] = zro

    pltpu.sync_copy(m_h, m_v)
    mv = m_v[...]
    _zero_rows1(zd, nacc)
    _zero_rows1(zd, dacc)
    plsc.subcore_barrier()

    @pl.loop(0, NBLK // 2)
    def _(i):
        be = _batch_base(i)
        cs = pltpu.async_copy(src_h.at[pl.ds(be, GB)], src_v, semL)
        cd = pltpu.async_copy(dst2_h.at[pl.ds(be // 128, GB // 128), :],
                              dst2_v, semL)
        cs.wait()
        cd.wait()
        for j in range(GB // 128):
            for m in range(8):
                dst1_v[pl.ds(j * 128 + m * 16, 16)] = (
                    dst2_v[j, pl.ds(m * 16, 16)])
        ga = pltpu.async_copy(hs_h.at[src_v], hsv, semA)
        gb = pltpu.async_copy(hd_h.at[dst1_v], hdv, semB)
        gh = pltpu.async_copy(h_h.at[src_v], hv, semR)
        ga.wait()
        gb.wait()
        gh.wait()
        for k in range(GB // 16):
            sl = pl.ds(k * 16, 16)
            a = hsv[sl] + hdv[sl]
            a = jnp.where(a > 0, a, 0.2 * a)
            e = jnp.exp(a - mv)
            ev[sl] = e
            pv[sl] = e * hv[sl]
        sc = []
        for j in range(GB // 128):
            sc.append(pltpu.async_copy(
                ev.at[pl.ds(j * 128, 128)],
                dacc.at[dst2_v.at[j]], semS, add=True))
            sc.append(pltpu.async_copy(
                pv.at[pl.ds(j * 128, 128)],
                nacc.at[dst2_v.at[j]], semS, add=True))
        for d in sc:
            d.wait()

    plsc.subcore_barrier()
    _flush1(nacc, num_out, c * NPAD)
    _flush1(dacc, den_out, c * NPAD)


def _sc_kernel(body, out_shapes, scratch):
    mesh = plsc.VectorSubcoreMesh(**_MESH)
    return pl.kernel(body, out_type=out_shapes, mesh=mesh,
                     scratch_types=scratch,
                     compiler_params=pltpu.CompilerParams(
                         use_tc_tiling_on_sc=False))


def _sc_deg(dst2, w):
    return _sc_kernel(
        _deg_body,
        [jax.ShapeDtypeStruct((2 * NPAD,), f32)],
        [pltpu.VMEM((GB,), f32), pltpu.VMEM((GB // 128, 128), i32),
         pltpu.VMEM((ZR,), f32), pltpu.VMEM_SHARED((NPAD,), f32),
         pltpu.SemaphoreType.DMA, pltpu.SemaphoreType.DMA],
    )(dst2, w)[0]


def _sc_norm(src, dst2, w, dinv):
    return _sc_kernel(
        _norm_body,
        [jax.ShapeDtypeStruct((EPAD,), f32)],
        [pltpu.VMEM((GB,), i32), pltpu.VMEM((GB,), i32),
         pltpu.VMEM((GB // 128, 128), i32),
         pltpu.VMEM((GB,), f32), pltpu.VMEM((GB,), f32),
         pltpu.VMEM((GB,), f32), pltpu.VMEM((GB,), f32),
         pltpu.SemaphoreType.DMA, pltpu.SemaphoreType.DMA,
         pltpu.SemaphoreType.DMA],
    )(src, dst2, w, dinv)[0]


def _sc_gat(hlo, hhi, hs, hd, mvec, src, dst2):
    outs = _sc_kernel(
        _gat_body,
        [jax.ShapeDtypeStruct((4 * NPAD, 16), f32),
         jax.ShapeDtypeStruct((2 * NPAD,), f32),
         jax.ShapeDtypeStruct((EPAD,), f32)],
        [pltpu.VMEM((GB,), i32), pltpu.VMEM((GB,), i32),
         pltpu.VMEM((GB // 128, 128), i32),
         pltpu.VMEM((GB,), f32), pltpu.VMEM((GB,), f32),
         pltpu.VMEM((GB,), f32), pltpu.VMEM((GB, 16), f32),
         pltpu.VMEM((ZR, 16), f32),
         pltpu.VMEM((ZR,), f32), pltpu.VMEM((16,), f32),
         pltpu.VMEM_SHARED((NPAD, 16), f32),
         pltpu.VMEM_SHARED((NPAD,), f32),
         pltpu.SemaphoreType.DMA, pltpu.SemaphoreType.DMA,
         pltpu.SemaphoreType.DMA, pltpu.SemaphoreType.DMA,
         pltpu.SemaphoreType.DMA],
    )(hlo, hhi, hs, hd, mvec, src, dst2)
    return outs[0], outs[1]


def _sc_gcn(tabs, src, dst2, norm):
    return _sc_kernel(
        _gcn_body,
        [jax.ShapeDtypeStruct((10 * NPAD, 16), f32)],
        [pltpu.VMEM((GB,), i32), pltpu.VMEM((GB // 128, 128), i32),
         pltpu.VMEM((GB,), f32), pltpu.VMEM((GB, 16), f32),
         pltpu.VMEM((ZR, 16), f32),
         pltpu.VMEM_SHARED((NPAD, 16), f32),
         pltpu.SemaphoreType.DMA, pltpu.SemaphoreType.DMA,
         pltpu.SemaphoreType.DMA],
    )(*tabs, src, dst2, norm)[0]


def _sc_gat1(hs, hd, h, mvec, src, dst2):
    outs = _sc_kernel(
        _gat1_body,
        [jax.ShapeDtypeStruct((2 * NPAD,), f32),
         jax.ShapeDtypeStruct((2 * NPAD,), f32)],
        [pltpu.VMEM((GB,), i32), pltpu.VMEM((GB,), i32),
         pltpu.VMEM((GB // 128, 128), i32),
         pltpu.VMEM((GB,), f32), pltpu.VMEM((GB,), f32),
         pltpu.VMEM((GB,), f32), pltpu.VMEM((GB,), f32),
         pltpu.VMEM((GB,), f32), pltpu.VMEM((ZR,), f32),
         pltpu.VMEM_SHARED((NPAD,), f32),
         pltpu.VMEM_SHARED((NPAD,), f32), pltpu.VMEM((16,), f32),
         pltpu.SemaphoreType.DMA, pltpu.SemaphoreType.DMA,
         pltpu.SemaphoreType.DMA, pltpu.SemaphoreType.DMA,
         pltpu.SemaphoreType.DMA],
    )(hs, hd, h, mvec, src, dst2)
    return outs[0], outs[1]


def _row_spec(d):
    return pl.BlockSpec((BR, d), lambda r: (r, 0))


def _full_spec(shape):
    nd = len(shape)
    return pl.BlockSpec(shape, lambda r: (0,) * nd)


def _lead_spec(lead, d):
    return pl.BlockSpec((lead, BR, d), lambda r: (0, r, 0))


def _tc_call(body, in_specs, out_shapes, out_specs):
    return pl.pallas_call(
        body, grid=(NG,), in_specs=in_specs,
        out_shape=[jax.ShapeDtypeStruct(s, f32) for s in out_shapes],
        out_specs=out_specs)


def _dgT(a, b):
    return lax.dot_general(a, b, (((1,), (1,)), ((), ())),
                           preferred_element_type=f32)


def _col_spec(d):
    return pl.BlockSpec((d, BR), lambda r: (0, r))


def _cvec_spec(lead):
    return pl.BlockSpec((lead, BR), lambda r: (0, r))


_VROW = pl.BlockSpec((1, BR), lambda r: (0, r))


def _tca_body(o0t_ref, w_ref, as_ref, ad_ref, hlo_ref, hhi_ref,
              hs_ref, hd_ref):
    h = lax.dot_general(o0t_ref[...], w_ref[...],
                        (((0,), (0,)), ((), ())),
                        preferred_element_type=f32)
    hlo_ref[...] = h[:, :16]
    hhi_ref[...] = h[:, 16:32]
    hs_ref[...] = _dgT(as_ref[...], h)
    hd_ref[...] = _dgT(ad_ref[...], h)


def _tca(o0t, W, a_s, a_d, k):
    return _tc_call(
        _tca_body,
        [_col_spec(k), _full_spec((k, HID)), _full_spec((1, HID)),
         _full_spec((1, HID))],
        [(NPAD, 16), (NPAD, 16), (1, NPAD), (1, NPAD)],
        [_row_spec(16), _row_spec(16), _VROW, _VROW])(
            o0t, W, a_s.reshape(1, HID), a_d.reshape(1, HID))


def _dinv_body(deg_ref, dinv_ref):
    d = deg_ref[...]
    dt = d[0:1] + d[1:2]
    dinv_ref[...] = jnp.where(dt > 0, lax.rsqrt(dt), 0.0)


def _dinv(deg2):
    return _tc_call(
        _dinv_body, [_cvec_spec(2)], [(1, NPAD)],
        [_VROW])(deg2.reshape(2, NPAD))[0]


def _tcb_body(num_ref, den_ref, b_ref, w_ref, as_ref, ad_ref, eye_ref,
              o1_ref, hlo_ref, hhi_ref, hs_ref, hd_ref):
    n = num_ref[...]
    dn = den_ref[...]
    numv = jnp.concatenate([n[0] + n[2], n[1] + n[3]], axis=1)
    dsum = dn[0:1] + dn[1:2]
    rcp = 1.0 / (dsum + 1e-16)
    rcp_col = lax.dot_general(eye_ref[...], rcp,
                              (((1,), (1,)), ((), ())),
                              preferred_element_type=f32)
    o1 = numv * rcp_col + b_ref[...]
    o1_ref[...] = o1
    h = jnp.dot(o1, w_ref[...], preferred_element_type=f32)
    hlo_ref[...] = h[:, :16]
    hhi_ref[...] = h[:, 16:32]
    hs_ref[...] = _dgT(as_ref[...], h)
    hd_ref[...] = _dgT(ad_ref[...], h)


def _tcb(num, den, b, W, a_s, a_d, eye):
    return _tc_call(
        _tcb_body,
        [_lead_spec(4, 16), _cvec_spec(2), _full_spec((1, HID)),
         _full_spec((HID, HID)), _full_spec((1, HID)),
         _full_spec((1, HID)), _full_spec((BR, BR))],
        [(NPAD, HID), (NPAD, 16), (NPAD, 16), (1, NPAD), (1, NPAD)],
        [_row_spec(HID), _row_spec(16), _row_spec(16), _VROW, _VROW])(
            num.reshape(4, NPAD, 16), den.reshape(2, NPAD),
            b.reshape(1, HID), W, a_s.reshape(1, HID),
            a_d.reshape(1, HID), eye)


def _tcc_body(num_ref, den_ref, b_ref, o0t_ref, o1_ref, wg0_ref,
              wg1_ref, wg2_ref, eye_ref, ht_ref):
    n = num_ref[...]
    dn = den_ref[...]
    numv = jnp.concatenate([n[0] + n[2], n[1] + n[3]], axis=1)
    rcp = 1.0 / (dn[0:1] + dn[1:2] + 1e-16)
    rcp_col = lax.dot_general(eye_ref[...], rcp,
                              (((1,), (1,)), ((), ())),
                              preferred_element_type=f32)
    o2 = numv * rcp_col + b_ref[...]
    ht = (lax.dot_general(o0t_ref[...], wg0_ref[...],
                          (((0,), (0,)), ((), ())),
                          preferred_element_type=f32)
          + jnp.dot(o1_ref[...], wg1_ref[...], preferred_element_type=f32)
          + jnp.dot(o2, wg2_ref[...], preferred_element_type=f32))
    ht_ref[...] = jnp.concatenate([ht, jnp.zeros((BR, 4), f32)], axis=1)


def _tcc(num, den, b, o0t, o1, Wg, eye):
    return _tc_call(
        _tcc_body,
        [_lead_spec(4, 16), _cvec_spec(2), _full_spec((1, HID)),
         _col_spec(T_PAST), _row_spec(HID), _full_spec((T_PAST, TEMP)),
         _full_spec((HID, TEMP)), _full_spec((HID, TEMP)),
         _full_spec((BR, BR))],
        [(NPAD, 80)],
        [_row_spec(80)])(
            num.reshape(4, NPAD, 16), den.reshape(2, NPAD),
            b.reshape(1, HID), o0t, o1, Wg[:T_PAST], Wg[T_PAST:44],
            Wg[44:], eye)


def _tcd_body(acc_ref, wk_ref, b_ref, out_ref):
    a = acc_ref[...]
    wk = wk_ref[...]
    tot = jnp.zeros((NPAD // 8 // NG, 128), f32) + b_ref[0, 0]
    for p in range(5):
        tot = tot + jnp.dot(a[p] + a[5 + p], wk[p],
                            preferred_element_type=f32)
    out_ref[...] = tot


def _tcd(accg, Wk3, bgw3):
    npk = NPAD // 8
    return _tc_call(
        _tcd_body,
        [pl.BlockSpec((10, npk // NG, 128), lambda r: (0, r, 0)),
         _full_spec((5, 128, 128)), _full_spec((1, 1))],
        [(npk, 128)],
        [pl.BlockSpec((npk // NG, 128), lambda r: (r, 0))])(
            accg.reshape(10, npk, 128), Wk3, bgw3.reshape(1, 1))


def _tce_body(num_ref, den_ref, b_ref, o0t_ref, yp_ref, o0n_ref):
    n = num_ref[...]
    dn = den_ref[...]
    yp = (n[0:1] + n[1:2]) / (dn[0:1] + dn[1:2] + 1e-16) + b_ref[0, 0]
    yp = jnp.clip(yp, 0.0, 90.0)
    yp_ref[...] = yp
    o0n_ref[...] = jnp.concatenate([o0t_ref[...][1:], yp], axis=0)


def _tce(num, den, b3, o0t):
    return _tc_call(
        _tce_body,
        [_cvec_spec(2), _cvec_spec(2), _full_spec((1, 1)),
         _col_spec(T_PAST)],
        [(1, NPAD), (T_PAST, NPAD)],
        [_VROW, _col_spec(T_PAST)])(
            num.reshape(2, NPAD), den.reshape(2, NPAD),
            b3.reshape(1, 1), o0t)


def _leaky(v):
    return jnp.where(v > 0, v, 0.2 * v)


def _mvec(hs, hd):
    m = _leaky(jnp.max(hs.reshape(NPAD)[:N]) + jnp.max(hd.reshape(NPAD)[:N]))
    return jnp.full((16,), m, f32)


def kernel(x, edge_index, edge_attr, y, W0, a_src0, a_dst0, b0,
           W1, a_src1, a_dst1, b1, Wg, bg, W3, a_src3, a_dst3, b3):
    t_future = y.shape[1]
    loop = jnp.arange(N, dtype=edge_index.dtype)
    padn = EPAD - EE
    src = jnp.concatenate(
        [edge_index[0], loop, jnp.zeros((padn,), edge_index.dtype)])
    dst = jnp.concatenate(
        [edge_index[1], loop, jnp.full((padn,), N, edge_index.dtype)])
    w = jnp.concatenate(
        [edge_attr, jnp.ones((N,), f32), jnp.zeros((padn,), f32)])

    dst2 = dst.reshape(EPAD // 128, 128)
    deg2 = _sc_deg(dst2, w)
    dinv = _dinv(deg2)
    norm = _sc_norm(src, dst2, w, dinv.reshape(NPAD))

    o0t = jnp.concatenate(
        [x, jnp.zeros((NPAD - N, T_PAST), f32)], axis=0).T
    eye = jnp.eye(BR, dtype=f32)
    w3pad = jnp.concatenate([W3[:, 0], jnp.zeros((4,), f32)])
    Wk3 = jnp.stack([
        jnp.kron(jnp.eye(8, dtype=f32),
                 w3pad[16 * p:16 * p + 16][:, None] * jnp.ones((1, 16), f32))
        for p in range(5)])
    bgw3 = jnp.dot(bg, W3[:, 0]).reshape(1, 1)
    preds = []
    for _step in range(t_future):
        hlo, hhi, hs0, hd0 = _tca(o0t, W0, a_src0, a_dst0, T_PAST)
        num0, den0 = _sc_gat(hlo, hhi, hs0.reshape(NPAD),
                             hd0.reshape(NPAD), _mvec(hs0, hd0),
                             src, dst2)
        o1, h1lo, h1hi, hs1, hd1 = _tcb(num0, den0, b0, W1,
                                        a_src1, a_dst1, eye)
        num1, den1 = _sc_gat(h1lo, h1hi, hs1.reshape(NPAD),
                             hd1.reshape(NPAD), _mvec(hs1, hd1),
                             src, dst2)
        ht80 = _tcc(num1, den1, b1, o0t, o1, Wg, eye)[0]
        tabs = [ht80[:, 16 * p:16 * p + 16] for p in range(5)]
        accg = _sc_gcn(tabs, src, dst2, norm)
        h3rep = _tcd(accg, Wk3, bgw3)[0]
        h3f = h3rep.reshape(NPAD, 16)[:, 0]
        hs3f = h3f * a_src3[0]
        hd3f = h3f * a_dst3[0]
        m3 = jnp.full((16,), _leaky(jnp.max(hs3f[:N]) + jnp.max(hd3f[:N])),
                      f32)
        num3, den3 = _sc_gat1(hs3f, hd3f, h3f, m3, src, dst2)
        yp, o0t = _tce(num3, den3, b3, o0t)
        preds.append(yp.reshape(NPAD, 1)[:N])

    return jnp.concatenate(preds, axis=1)
